# Initial kernel scaffold; baseline (speedup 1.0000x reference)
import jax
import jax.numpy as jnp
from jax.experimental import pallas as pl


def kernel(node_feat, edge_feat, edge_idx, W1a, b1a, W1b, b1b, W2a, b2a, W2b, b2b, W3a, b3a, W3b, b3b, W4a, b4a, W4b, b4b, W5a, b5a, W5b, b5b, W6a, b6a, W6b, b6b, Wl, bl):
    raise NotImplementedError("write your pallas kernel here")


# speedup vs baseline: 2.7528x; 2.7528x over previous
import functools

import jax
import jax.numpy as jnp
from jax import lax
from jax.experimental import pallas as pl
from jax.experimental.pallas import tpu as pltpu
from jax.experimental.pallas import tpu_sc as plsc

_NC, _NS, _L = 2, 16, 16
_NW = _NC * _NS
_F32 = jnp.float32


def _mesh():
    return plsc.VectorSubcoreMesh(core_axis_name="c", subcore_axis_name="s")


@functools.lru_cache(maxsize=None)
def _sc_counts(E, Npad, C, W):
    Ew = E // _NW
    nk = Ew // C
    rpt = Npad // _NS

    @functools.partial(
        pl.kernel,
        mesh=_mesh(),
        out_type=jax.ShapeDtypeStruct((_NC * Npad, W), _F32),
        scratch_types=[
            pltpu.VMEM((C,), jnp.int32),
            pltpu.VMEM((C, W), _F32),
            pltpu.VMEM_SHARED((Npad, W), _F32),
        ],
    )
    def k(row, ones, zeros, out, idxv, onesv, acc):
        c = lax.axis_index("c")
        s = lax.axis_index("s")
        t = c * _NS + s
        pltpu.sync_copy(zeros, acc.at[pl.ds(s * rpt, rpt)])
        pltpu.sync_copy(ones, onesv)
        plsc.subcore_barrier()

        def body(kk, carry):
            base = t * Ew + kk * C
            pltpu.sync_copy(row.at[pl.ds(base, C)], idxv)
            pltpu.sync_copy(onesv, acc.at[idxv], add=True)
            return carry

        lax.fori_loop(0, nk, body, 0)
        plsc.subcore_barrier()
        pltpu.sync_copy(acc.at[pl.ds(s * rpt, rpt)],
                        out.at[pl.ds(c * Npad + s * rpt, rpt)])

    return k


@functools.lru_cache(maxsize=None)
def _sc_gather(E, Npad, F, C):
    Ew = E // _NW
    nk = Ew // C

    @functools.partial(
        pl.kernel,
        mesh=_mesh(),
        out_type=(jax.ShapeDtypeStruct((E, F), _F32),
                  jax.ShapeDtypeStruct((E, F), _F32)),
        scratch_types=[
            pltpu.VMEM((C,), jnp.int32),
            pltpu.VMEM((C,), jnp.int32),
            pltpu.VMEM((C, F), _F32),
            pltpu.VMEM((C, F), _F32),
            pltpu.SemaphoreType.DMA,
            pltpu.SemaphoreType.DMA,
        ],
    )
    def k(ta, tb, row, col, gr, gc, idxr, idxc, bufr, bufc, sem1, sem2):
        c = lax.axis_index("c")
        s = lax.axis_index("s")
        t = c * _NS + s

        def body(kk, carry):
            base = t * Ew + kk * C
            pltpu.sync_copy(row.at[pl.ds(base, C)], idxr)
            pltpu.sync_copy(col.at[pl.ds(base, C)], idxc)
            cp1 = pltpu.async_copy(ta.at[idxr], bufr, sem1)
            cp2 = pltpu.async_copy(tb.at[idxc], bufc, sem2)
            cp1.wait()
            cp2.wait()
            pltpu.sync_copy(bufr, gr.at[pl.ds(base, C)])
            pltpu.sync_copy(bufc, gc.at[pl.ds(base, C)])
            return carry

        lax.fori_loop(0, nk, body, 0)

    return k


@functools.lru_cache(maxsize=None)
def _sc_scatter(E, Npad, F, C):
    Ew = E // _NW
    nk = Ew // C
    rpt = Npad // _NS

    @functools.partial(
        pl.kernel,
        mesh=_mesh(),
        out_type=jax.ShapeDtypeStruct((_NC * Npad, F), _F32),
        scratch_types=[
            pltpu.VMEM((C,), jnp.int32),
            pltpu.VMEM((C, F), _F32),
            pltpu.VMEM_SHARED((Npad, F), _F32),
        ],
    )
    def k(eo, row, zeros, out, idxv, buf, acc):
        c = lax.axis_index("c")
        s = lax.axis_index("s")
        t = c * _NS + s
        pltpu.sync_copy(zeros, acc.at[pl.ds(s * rpt, rpt)])
        plsc.subcore_barrier()

        def body(kk, carry):
            base = t * Ew + kk * C
            pltpu.sync_copy(row.at[pl.ds(base, C)], idxv)
            pltpu.sync_copy(eo.at[pl.ds(base, C)], buf)
            pltpu.sync_copy(buf, acc.at[idxv], add=True)
            return carry

        lax.fori_loop(0, nk, body, 0)
        plsc.subcore_barrier()
        pltpu.sync_copy(acc.at[pl.ds(s * rpt, rpt)],
                        out.at[pl.ds(c * Npad + s * rpt, rpt)])

    return k


@functools.lru_cache(maxsize=None)
def _tc_proj1(Npad, D, F, Bn):
    def body(x, wi, wj, ta, tb):
        xv = x[...]
        ta[...] = jnp.dot(xv, wi[...], preferred_element_type=_F32)
        tb[...] = jnp.dot(xv, wj[...], preferred_element_type=_F32)

    return pl.pallas_call(
        body,
        grid=(Npad // Bn,),
        in_specs=[
            pl.BlockSpec((Bn, D), lambda i: (i, 0)),
            pl.BlockSpec((D, F), lambda i: (0, 0)),
            pl.BlockSpec((D, F), lambda i: (0, 0)),
        ],
        out_specs=[pl.BlockSpec((Bn, F), lambda i: (i, 0))] * 2,
        out_shape=[jax.ShapeDtypeStruct((Npad, F), _F32)] * 2,
        compiler_params=pltpu.CompilerParams(
            dimension_semantics=("parallel",)),
    )


@functools.lru_cache(maxsize=None)
def _tc_node(Npad, F, CW, has_prev, Bn):
    nb = Npad // Bn

    def body(*refs):
        if has_prev:
            (p0, p1, c0, c1, xp, wi1, wi2, wj1, wj2, xo, ta, tb) = refs
        else:
            (p0, p1, c0, c1, wi1, wj1, xo, ta, tb) = refs
        cnt = c0[:, :1] + c1[:, :1]
        inv = 1.0 / jnp.maximum(cnt, 1.0)
        x = jnp.maximum((p0[...] + p1[...]) * inv, 0.0)
        xo[...] = x
        tav = jnp.dot(x, wi1[...], preferred_element_type=_F32)
        tbv = jnp.dot(x, wj1[...], preferred_element_type=_F32)
        if has_prev:
            xpv = xp[...]
            tav = tav + jnp.dot(xpv, wi2[...], preferred_element_type=_F32)
            tbv = tbv + jnp.dot(xpv, wj2[...], preferred_element_type=_F32)
        ta[...] = tav
        tb[...] = tbv

    bspec = pl.BlockSpec((Bn, F), lambda i: (i, 0))
    p0s = pl.BlockSpec((Bn, F), lambda i: (i, 0))
    p1s = pl.BlockSpec((Bn, F), lambda i: (nb + i, 0))
    c0s = pl.BlockSpec((Bn, CW), lambda i: (i, 0))
    c1s = pl.BlockSpec((Bn, CW), lambda i: (nb + i, 0))
    wspec = pl.BlockSpec((F, F), lambda i: (0, 0))
    in_specs = [p0s, p1s, c0s, c1s]
    if has_prev:
        in_specs += [bspec, wspec, wspec, wspec, wspec]
    else:
        in_specs += [wspec, wspec]
    return pl.pallas_call(
        body,
        grid=(nb,),
        in_specs=in_specs,
        out_specs=[bspec] * 3,
        out_shape=[jax.ShapeDtypeStruct((Npad, F), _F32)] * 3,
        compiler_params=pltpu.CompilerParams(
            dimension_semantics=("parallel",)),
    )


@functools.lru_cache(maxsize=None)
def _tc_node_final(Npad, F, CW, Bn):
    nb = Npad // Bn

    def body(p0, p1, c0, c1, xo):
        cnt = c0[:, :1] + c1[:, :1]
        inv = 1.0 / jnp.maximum(cnt, 1.0)
        xo[...] = jnp.maximum((p0[...] + p1[...]) * inv, 0.0)

    return pl.pallas_call(
        body,
        grid=(nb,),
        in_specs=[
            pl.BlockSpec((Bn, F), lambda i: (i, 0)),
            pl.BlockSpec((Bn, F), lambda i: (nb + i, 0)),
            pl.BlockSpec((Bn, CW), lambda i: (i, 0)),
            pl.BlockSpec((Bn, CW), lambda i: (nb + i, 0)),
        ],
        out_specs=pl.BlockSpec((Bn, F), lambda i: (i, 0)),
        out_shape=jax.ShapeDtypeStruct((Npad, F), _F32),
        compiler_params=pltpu.CompilerParams(
            dimension_semantics=("parallel",)),
    )


@functools.lru_cache(maxsize=None)
def _tc_edge(E, F, DE1, nparts, relu_e, final, Be):
    def body(*refs):
        i = 0
        gr, gc = refs[0], refs[1]
        i = 2
        e1 = refs[i]; i += 1
        e2 = refs[i] if nparts == 2 else None
        if nparts == 2:
            i += 1
        we1 = refs[i]; i += 1
        we2 = refs[i] if nparts == 2 else None
        if nparts == 2:
            i += 1
        ba, wb, bb = refs[i], refs[i + 1], refs[i + 2]
        i += 3
        if final:
            wl, bl = refs[i], refs[i + 1]
            i += 2
        eo_out = refs[i]; i += 1
        if final:
            e6_out, out_out = refs[i], refs[i + 1]

        a = gr[...] + gc[...] + ba[...]
        ev1 = e1[...]
        if relu_e:
            ev1 = jnp.maximum(ev1, 0.0)
        a = a + jnp.dot(ev1, we1[...], preferred_element_type=_F32)
        if nparts == 2:
            ev2 = jnp.maximum(e2[...], 0.0)
            a = a + jnp.dot(ev2, we2[...], preferred_element_type=_F32)
        h = jnp.maximum(a, 0.0)
        eo = jnp.dot(h, wb[...], preferred_element_type=_F32) + bb[...]
        eo_out[...] = eo
        if final:
            e6 = jnp.maximum(eo, 0.0)
            e6_out[...] = e6
            out_out[...] = jnp.dot(e6, wl[...],
                                   preferred_element_type=_F32) + bl[...]

    bspec = pl.BlockSpec((Be, F), lambda i: (i, 0))
    in_specs = [bspec, bspec, pl.BlockSpec((Be, DE1), lambda i: (i, 0))]
    if nparts == 2:
        in_specs.append(bspec)
    in_specs.append(pl.BlockSpec((DE1, F), lambda i: (0, 0)))
    if nparts == 2:
        in_specs.append(pl.BlockSpec((F, F), lambda i: (0, 0)))
    in_specs += [
        pl.BlockSpec((1, F), lambda i: (0, 0)),
        pl.BlockSpec((F, F), lambda i: (0, 0)),
        pl.BlockSpec((1, F), lambda i: (0, 0)),
    ]
    out_specs = [bspec]
    out_shape = [jax.ShapeDtypeStruct((E, F), _F32)]
    if final:
        in_specs += [
            pl.BlockSpec((F, 1), lambda i: (0, 0)),
            pl.BlockSpec((1, 1), lambda i: (0, 0)),
        ]
        out_specs += [bspec, pl.BlockSpec((Be, 1), lambda i: (i, 0))]
        out_shape += [jax.ShapeDtypeStruct((E, F), _F32),
                      jax.ShapeDtypeStruct((E, 1), _F32)]
    return pl.pallas_call(
        body,
        grid=(E // Be,),
        in_specs=in_specs,
        out_specs=out_specs,
        out_shape=out_shape,
        compiler_params=pltpu.CompilerParams(
            dimension_semantics=("parallel",)),
    )


def kernel(node_feat, edge_feat, edge_idx,
           W1a, b1a, W1b, b1b, W2a, b2a, W2b, b2b,
           W3a, b3a, W3b, b3b, W4a, b4a, W4b, b4b,
           W5a, b5a, W5b, b5b, W6a, b6a, W6b, b6b,
           Wl, bl):
    N, D = node_feat.shape
    E, DE = edge_feat.shape
    F = W1b.shape[0]
    Npad = -(-N // 1024) * 1024
    Bn = 1024
    Be = 2000
    C = 80
    CW = 128
    assert E % (_NW * C) == 0 and E % Be == 0 and Npad % (_NS * 8) == 0

    row = edge_idx[0]
    col = edge_idx[1]
    x0 = jnp.pad(node_feat, ((0, Npad - N), (0, 0)))
    rpt = Npad // _NS
    zeros_c = jnp.zeros((rpt, CW), _F32)
    zeros_f = jnp.zeros((rpt, F), _F32)
    ones_c = jnp.ones((C, CW), _F32)

    gather = _sc_gather(E, Npad, F, C)
    scatter = _sc_scatter(E, Npad, F, C)

    cnt = _sc_counts(E, Npad, C, CW)(row, ones_c, zeros_c)

    ta, tb = _tc_proj1(Npad, D, F, Bn)(x0, W1a[:D], W1a[D:2 * D])
    gr, gc = gather(ta, tb, row, col)
    eo1 = _tc_edge(E, F, DE, 1, False, False, Be)(
        gr, gc, edge_feat, W1a[2 * D:], b1a.reshape(1, F), W1b,
        b1b.reshape(1, F))[0]
    p = scatter(eo1, row, zeros_f)

    x1, ta, tb = _tc_node(Npad, F, CW, False, Bn)(
        p, p, cnt, cnt, W2a[:F], W2a[F:2 * F])
    gr, gc = gather(ta, tb, row, col)
    eo2 = _tc_edge(E, F, F, 1, True, False, Be)(
        gr, gc, eo1, W2a[2 * F:], b2a.reshape(1, F), W2b,
        b2b.reshape(1, F))[0]
    p = scatter(eo2, row, zeros_f)

    x_prev = x1
    eo_prev2, eo_prev = eo1, eo2
    e6 = outv = None
    for Wa, ba, Wb, bb in ((W3a, b3a, W3b, b3b), (W4a, b4a, W4b, b4b),
                           (W5a, b5a, W5b, b5b), (W6a, b6a, W6b, b6b)):
        final = Wa is W6a
        x_cur, ta, tb = _tc_node(Npad, F, CW, True, Bn)(
            p, p, cnt, cnt, x_prev,
            Wa[:F], Wa[F:2 * F], Wa[2 * F:3 * F], Wa[3 * F:4 * F])
        gr, gc = gather(ta, tb, row, col)
        edge_fn = _tc_edge(E, F, F, 2, True, final, Be)
        args = (gr, gc, eo_prev, eo_prev2, Wa[4 * F:5 * F], Wa[5 * F:6 * F],
                ba.reshape(1, F), Wb, bb.reshape(1, F))
        if final:
            eo, e6, outv = edge_fn(*args, Wl, bl.reshape(1, 1))
        else:
            eo = edge_fn(*args)[0]
        p = scatter(eo, row, zeros_f)
        x_prev, eo_prev2, eo_prev = x_cur, eo_prev, eo

    x6p = _tc_node_final(Npad, F, CW, Bn)(p, p, cnt, cnt)
    return (outv, x6p[:N], e6)


# speedup vs baseline: 3.5154x; 1.2770x over previous
import functools

import jax
import jax.numpy as jnp
from jax import lax
from jax.experimental import pallas as pl
from jax.experimental.pallas import tpu as pltpu
from jax.experimental.pallas import tpu_sc as plsc

_NC, _NS, _L = 2, 16, 16
_NW = _NC * _NS
_F32 = jnp.float32


def _mesh():
    return plsc.VectorSubcoreMesh(core_axis_name="c", subcore_axis_name="s")


@functools.lru_cache(maxsize=None)
def _sc_counts(E, Npad, C, W):
    Ew = E // _NW
    nk = Ew // C
    rpt = Npad // _NS

    @functools.partial(
        pl.kernel,
        mesh=_mesh(),
        out_type=jax.ShapeDtypeStruct((_NC * Npad, W), _F32),
        scratch_types=[
            pltpu.VMEM((nk, C), jnp.int32),
            pltpu.VMEM((C, W), _F32),
            pltpu.VMEM_SHARED((Npad, W), _F32),
        ],
    )
    def k(row3, ones, zeros, out, idxv, onesv, acc):
        c = lax.axis_index("c")
        s = lax.axis_index("s")
        t = c * _NS + s
        pltpu.sync_copy(zeros, acc.at[pl.ds(s * rpt, rpt)])
        pltpu.sync_copy(ones, onesv)
        pltpu.sync_copy(row3.at[t], idxv)
        plsc.subcore_barrier()

        def body(kk, carry):
            pltpu.sync_copy(onesv, acc.at[idxv.at[kk]], add=True)
            return carry

        lax.fori_loop(0, nk, body, 0)
        plsc.subcore_barrier()
        pltpu.sync_copy(acc.at[pl.ds(s * rpt, rpt)],
                        out.at[pl.ds(c * Npad + s * rpt, rpt)])

    return k


@functools.lru_cache(maxsize=None)
def _sc_gather(E, Npad, F, C, NB=2):
    Ew = E // _NW
    nk = Ew // C
    nr = nk // NB

    @functools.partial(
        pl.kernel,
        mesh=_mesh(),
        out_type=(jax.ShapeDtypeStruct((E, F), _F32),
                  jax.ShapeDtypeStruct((E, F), _F32)),
        scratch_types=[
            pltpu.VMEM((nk, C), jnp.int32),
            pltpu.VMEM((nk, C), jnp.int32),
        ] + [pltpu.VMEM((C, F), _F32)] * (2 * NB)
          + [pltpu.SemaphoreType.DMA] * (4 * NB),
    )
    def k(ta, tb, row3, col3, gr, gc, idxr, idxc, *bufsems):
        bufr = bufsems[:NB]
        bufc = bufsems[NB:2 * NB]
        sems = bufsems[2 * NB:]
        grs, gcs, srs, scs = (sems[:NB], sems[NB:2 * NB],
                              sems[2 * NB:3 * NB], sems[3 * NB:])
        c = lax.axis_index("c")
        s = lax.axis_index("s")
        t = c * _NS + s
        pltpu.sync_copy(row3.at[t], idxr)
        pltpu.sync_copy(col3.at[t], idxc)
        for b in range(NB):
            pltpu.async_copy(ta.at[idxr.at[b]], bufr[b], grs[b])
            pltpu.async_copy(tb.at[idxc.at[b]], bufc[b], gcs[b])

        def body(g, carry):
            for b in range(NB):
                kk = g * NB + b
                base = t * Ew + kk * C
                pltpu.make_async_copy(ta.at[idxr.at[kk]], bufr[b],
                                      grs[b]).wait()
                pltpu.make_async_copy(tb.at[idxc.at[kk]], bufc[b],
                                      gcs[b]).wait()
                pltpu.async_copy(bufr[b], gr.at[pl.ds(base, C)], srs[b])
                pltpu.async_copy(bufc[b], gc.at[pl.ds(base, C)], scs[b])
            for b in range(NB):
                kk = g * NB + b
                base = t * Ew + kk * C

                @pl.when(g + 1 < nr)
                def _():
                    pltpu.make_async_copy(bufr[b], gr.at[pl.ds(base, C)],
                                          srs[b]).wait()
                    pltpu.make_async_copy(bufc[b], gc.at[pl.ds(base, C)],
                                          scs[b]).wait()
                    pltpu.async_copy(ta.at[idxr.at[kk + NB]], bufr[b], grs[b])
                    pltpu.async_copy(tb.at[idxc.at[kk + NB]], bufc[b], gcs[b])
            return carry

        lax.fori_loop(0, nr, body, 0)
        for b in range(NB):
            kk = nk - NB + b
            base = t * Ew + kk * C
            pltpu.make_async_copy(bufr[b], gr.at[pl.ds(base, C)],
                                  srs[b]).wait()
            pltpu.make_async_copy(bufc[b], gc.at[pl.ds(base, C)],
                                  scs[b]).wait()

    return k


@functools.lru_cache(maxsize=None)
def _sc_scatter(E, Npad, F, C, NB=2):
    Ew = E // _NW
    nk = Ew // C
    nr = nk // NB
    rpt = Npad // _NS

    @functools.partial(
        pl.kernel,
        mesh=_mesh(),
        out_type=jax.ShapeDtypeStruct((_NC * Npad, F), _F32),
        scratch_types=[
            pltpu.VMEM((nk, C), jnp.int32),
            pltpu.VMEM_SHARED((Npad, F), _F32),
        ] + [pltpu.VMEM((C, F), _F32)] * NB
          + [pltpu.SemaphoreType.DMA] * (2 * NB),
    )
    def k(eo, row3, zeros, out, idxv, acc, *bufsems):
        buf = bufsems[:NB]
        lds = bufsems[NB:2 * NB]
        scs = bufsems[2 * NB:]
        c = lax.axis_index("c")
        s = lax.axis_index("s")
        t = c * _NS + s
        pltpu.sync_copy(zeros, acc.at[pl.ds(s * rpt, rpt)])
        pltpu.sync_copy(row3.at[t], idxv)
        plsc.subcore_barrier()
        for b in range(NB):
            pltpu.async_copy(eo.at[pl.ds(t * Ew + b * C, C)], buf[b], lds[b])

        def body(g, carry):
            for b in range(NB):
                kk = g * NB + b
                base = t * Ew + kk * C
                pltpu.make_async_copy(eo.at[pl.ds(base, C)], buf[b],
                                      lds[b]).wait()
                pltpu.async_copy(buf[b], acc.at[idxv.at[kk]], scs[b],
                                 add=True)
            for b in range(NB):
                kk = g * NB + b

                @pl.when(g + 1 < nr)
                def _():
                    pltpu.make_async_copy(buf[b], acc.at[idxv.at[kk]],
                                          scs[b]).wait()
                    pltpu.async_copy(eo.at[pl.ds(t * Ew + (kk + NB) * C, C)],
                                     buf[b], lds[b])
            return carry

        lax.fori_loop(0, nr, body, 0)
        for b in range(NB):
            kk = nk - NB + b
            pltpu.make_async_copy(buf[b], acc.at[idxv.at[kk]], scs[b]).wait()
        plsc.subcore_barrier()
        pltpu.sync_copy(acc.at[pl.ds(s * rpt, rpt)],
                        out.at[pl.ds(c * Npad + s * rpt, rpt)])

    return k


@functools.lru_cache(maxsize=None)
def _tc_proj1(Npad, D, F, Bn):
    def body(x, wi, wj, ta, tb):
        xv = x[...]
        ta[...] = jnp.dot(xv, wi[...], preferred_element_type=_F32)
        tb[...] = jnp.dot(xv, wj[...], preferred_element_type=_F32)

    return pl.pallas_call(
        body,
        grid=(Npad // Bn,),
        in_specs=[
            pl.BlockSpec((Bn, D), lambda i: (i, 0)),
            pl.BlockSpec((D, F), lambda i: (0, 0)),
            pl.BlockSpec((D, F), lambda i: (0, 0)),
        ],
        out_specs=[pl.BlockSpec((Bn, F), lambda i: (i, 0))] * 2,
        out_shape=[jax.ShapeDtypeStruct((Npad, F), _F32)] * 2,
        compiler_params=pltpu.CompilerParams(
            dimension_semantics=("parallel",)),
    )


@functools.lru_cache(maxsize=None)
def _tc_node(Npad, F, CW, has_prev, Bn):
    nb = Npad // Bn

    def body(*refs):
        if has_prev:
            (p0, p1, c0, c1, xp, wi1, wi2, wj1, wj2, xo, ta, tb) = refs
        else:
            (p0, p1, c0, c1, wi1, wj1, xo, ta, tb) = refs
        cnt = c0[:, :1] + c1[:, :1]
        inv = 1.0 / jnp.maximum(cnt, 1.0)
        x = jnp.maximum((p0[...] + p1[...]) * inv, 0.0)
        xo[...] = x
        tav = jnp.dot(x, wi1[...], preferred_element_type=_F32)
        tbv = jnp.dot(x, wj1[...], preferred_element_type=_F32)
        if has_prev:
            xpv = xp[...]
            tav = tav + jnp.dot(xpv, wi2[...], preferred_element_type=_F32)
            tbv = tbv + jnp.dot(xpv, wj2[...], preferred_element_type=_F32)
        ta[...] = tav
        tb[...] = tbv

    bspec = pl.BlockSpec((Bn, F), lambda i: (i, 0))
    p0s = pl.BlockSpec((Bn, F), lambda i: (i, 0))
    p1s = pl.BlockSpec((Bn, F), lambda i: (nb + i, 0))
    c0s = pl.BlockSpec((Bn, CW), lambda i: (i, 0))
    c1s = pl.BlockSpec((Bn, CW), lambda i: (nb + i, 0))
    wspec = pl.BlockSpec((F, F), lambda i: (0, 0))
    in_specs = [p0s, p1s, c0s, c1s]
    if has_prev:
        in_specs += [bspec, wspec, wspec, wspec, wspec]
    else:
        in_specs += [wspec, wspec]
    return pl.pallas_call(
        body,
        grid=(nb,),
        in_specs=in_specs,
        out_specs=[bspec] * 3,
        out_shape=[jax.ShapeDtypeStruct((Npad, F), _F32)] * 3,
        compiler_params=pltpu.CompilerParams(
            dimension_semantics=("parallel",)),
    )


@functools.lru_cache(maxsize=None)
def _tc_node_final(Npad, F, CW, Bn):
    nb = Npad // Bn

    def body(p0, p1, c0, c1, xo):
        cnt = c0[:, :1] + c1[:, :1]
        inv = 1.0 / jnp.maximum(cnt, 1.0)
        xo[...] = jnp.maximum((p0[...] + p1[...]) * inv, 0.0)

    return pl.pallas_call(
        body,
        grid=(nb,),
        in_specs=[
            pl.BlockSpec((Bn, F), lambda i: (i, 0)),
            pl.BlockSpec((Bn, F), lambda i: (nb + i, 0)),
            pl.BlockSpec((Bn, CW), lambda i: (i, 0)),
            pl.BlockSpec((Bn, CW), lambda i: (nb + i, 0)),
        ],
        out_specs=pl.BlockSpec((Bn, F), lambda i: (i, 0)),
        out_shape=jax.ShapeDtypeStruct((Npad, F), _F32),
        compiler_params=pltpu.CompilerParams(
            dimension_semantics=("parallel",)),
    )


@functools.lru_cache(maxsize=None)
def _tc_edge(E, F, DE1, nparts, relu_e, final, Be):
    def body(*refs):
        i = 0
        gr, gc = refs[0], refs[1]
        i = 2
        e1 = refs[i]; i += 1
        e2 = refs[i] if nparts == 2 else None
        if nparts == 2:
            i += 1
        we1 = refs[i]; i += 1
        we2 = refs[i] if nparts == 2 else None
        if nparts == 2:
            i += 1
        ba, wb, bb = refs[i], refs[i + 1], refs[i + 2]
        i += 3
        if final:
            wl, bl = refs[i], refs[i + 1]
            i += 2
        eo_out = refs[i]; i += 1
        if final:
            e6_out, out_out = refs[i], refs[i + 1]

        a = gr[...] + gc[...] + ba[...]
        ev1 = e1[...]
        if relu_e:
            ev1 = jnp.maximum(ev1, 0.0)
        a = a + jnp.dot(ev1, we1[...], preferred_element_type=_F32)
        if nparts == 2:
            ev2 = jnp.maximum(e2[...], 0.0)
            a = a + jnp.dot(ev2, we2[...], preferred_element_type=_F32)
        h = jnp.maximum(a, 0.0)
        eo = jnp.dot(h, wb[...], preferred_element_type=_F32) + bb[...]
        eo_out[...] = eo
        if final:
            e6 = jnp.maximum(eo, 0.0)
            e6_out[...] = e6
            out_out[...] = jnp.dot(e6, wl[...],
                                   preferred_element_type=_F32) + bl[...]

    bspec = pl.BlockSpec((Be, F), lambda i: (i, 0))
    in_specs = [bspec, bspec, pl.BlockSpec((Be, DE1), lambda i: (i, 0))]
    if nparts == 2:
        in_specs.append(bspec)
    in_specs.append(pl.BlockSpec((DE1, F), lambda i: (0, 0)))
    if nparts == 2:
        in_specs.append(pl.BlockSpec((F, F), lambda i: (0, 0)))
    in_specs += [
        pl.BlockSpec((1, F), lambda i: (0, 0)),
        pl.BlockSpec((F, F), lambda i: (0, 0)),
        pl.BlockSpec((1, F), lambda i: (0, 0)),
    ]
    out_specs = [bspec]
    out_shape = [jax.ShapeDtypeStruct((E, F), _F32)]
    if final:
        in_specs += [
            pl.BlockSpec((F, 1), lambda i: (0, 0)),
            pl.BlockSpec((1, 1), lambda i: (0, 0)),
        ]
        out_specs += [bspec, pl.BlockSpec((Be, 1), lambda i: (i, 0))]
        out_shape += [jax.ShapeDtypeStruct((E, F), _F32),
                      jax.ShapeDtypeStruct((E, 1), _F32)]
    return pl.pallas_call(
        body,
        grid=(E // Be,),
        in_specs=in_specs,
        out_specs=out_specs,
        out_shape=out_shape,
        compiler_params=pltpu.CompilerParams(
            dimension_semantics=("parallel",)),
    )


def kernel(node_feat, edge_feat, edge_idx,
           W1a, b1a, W1b, b1b, W2a, b2a, W2b, b2b,
           W3a, b3a, W3b, b3b, W4a, b4a, W4b, b4b,
           W5a, b5a, W5b, b5b, W6a, b6a, W6b, b6b,
           Wl, bl):
    N, D = node_feat.shape
    E, DE = edge_feat.shape
    F = W1b.shape[0]
    Npad = -(-N // 1024) * 1024
    Bn = 1024
    Be = 2000
    C = 40
    CW = 128
    assert E % (_NW * C) == 0 and E % Be == 0 and Npad % (_NS * 8) == 0
    nk = E // (_NW * C)

    row = edge_idx[0]
    col = edge_idx[1]
    row3 = row.reshape(_NW, nk, C)
    col3 = col.reshape(_NW, nk, C)
    x0 = jnp.pad(node_feat, ((0, Npad - N), (0, 0)))
    rpt = Npad // _NS
    zeros_c = jnp.zeros((rpt, CW), _F32)
    zeros_f = jnp.zeros((rpt, F), _F32)
    ones_c = jnp.ones((C, CW), _F32)

    def gather(ta, tb, r3, c3):
        return _sc_gather(E, Npad, F, C)(ta, tb, r3, c3)

    def scatter(eo, r3, z):
        return _sc_scatter(E, Npad, F, C)(eo, r3, z)

    cnt = _sc_counts(E, Npad, C, CW)(row3, ones_c, zeros_c)

    ta, tb = _tc_proj1(Npad, D, F, Bn)(x0, W1a[:D], W1a[D:2 * D])
    gr, gc = gather(ta, tb, row3, col3)
    eo1 = _tc_edge(E, F, DE, 1, False, False, Be)(
        gr, gc, edge_feat, W1a[2 * D:], b1a.reshape(1, F), W1b,
        b1b.reshape(1, F))[0]
    p = scatter(eo1, row3, zeros_f)

    x1, ta, tb = _tc_node(Npad, F, CW, False, Bn)(
        p, p, cnt, cnt, W2a[:F], W2a[F:2 * F])
    gr, gc = gather(ta, tb, row3, col3)
    eo2 = _tc_edge(E, F, F, 1, True, False, Be)(
        gr, gc, eo1, W2a[2 * F:], b2a.reshape(1, F), W2b,
        b2b.reshape(1, F))[0]
    p = scatter(eo2, row3, zeros_f)

    x_prev = x1
    eo_prev2, eo_prev = eo1, eo2
    e6 = outv = None
    for Wa, ba, Wb, bb in ((W3a, b3a, W3b, b3b), (W4a, b4a, W4b, b4b),
                           (W5a, b5a, W5b, b5b), (W6a, b6a, W6b, b6b)):
        final = Wa is W6a
        x_cur, ta, tb = _tc_node(Npad, F, CW, True, Bn)(
            p, p, cnt, cnt, x_prev,
            Wa[:F], Wa[F:2 * F], Wa[2 * F:3 * F], Wa[3 * F:4 * F])
        gr, gc = gather(ta, tb, row3, col3)
        edge_fn = _tc_edge(E, F, F, 2, True, final, Be)
        args = (gr, gc, eo_prev, eo_prev2, Wa[4 * F:5 * F], Wa[5 * F:6 * F],
                ba.reshape(1, F), Wb, bb.reshape(1, F))
        if final:
            eo, e6, outv = edge_fn(*args, Wl, bl.reshape(1, 1))
        else:
            eo = edge_fn(*args)[0]
        p = scatter(eo, row3, zeros_f)
        x_prev, eo_prev2, eo_prev = x_cur, eo_prev, eo

    x6p = _tc_node_final(Npad, F, CW, Bn)(p, p, cnt, cnt)
    return (outv, x6p[:N], e6)


# speedup vs baseline: 3.7875x; 1.0774x over previous
import functools

import jax
import jax.numpy as jnp
from jax import lax
from jax.experimental import pallas as pl
from jax.experimental.pallas import tpu as pltpu
from jax.experimental.pallas import tpu_sc as plsc

_NC, _NS, _L = 2, 16, 16
_NW = _NC * _NS
_F32 = jnp.float32


def _mesh():
    return plsc.VectorSubcoreMesh(core_axis_name="c", subcore_axis_name="s")


@functools.lru_cache(maxsize=None)
def _sc_counts(E, Npad, C, W):
    Ew = E // _NW
    nk = Ew // C
    rpt = Npad // _NS

    @functools.partial(
        pl.kernel,
        mesh=_mesh(),
        out_type=jax.ShapeDtypeStruct((_NC * Npad, W), _F32),
        scratch_types=[
            pltpu.VMEM((nk, C), jnp.int32),
            pltpu.VMEM((C, W), _F32),
            pltpu.VMEM_SHARED((Npad, W), _F32),
        ],
    )
    def k(row3, ones, zeros, out, idxv, onesv, acc):
        c = lax.axis_index("c")
        s = lax.axis_index("s")
        t = c * _NS + s
        pltpu.sync_copy(zeros, acc.at[pl.ds(s * rpt, rpt)])
        pltpu.sync_copy(ones, onesv)
        pltpu.sync_copy(row3.at[t], idxv)
        plsc.subcore_barrier()

        def body(kk, carry):
            pltpu.sync_copy(onesv, acc.at[idxv.at[kk]], add=True)
            return carry

        lax.fori_loop(0, nk, body, 0)
        plsc.subcore_barrier()
        pltpu.sync_copy(acc.at[pl.ds(s * rpt, rpt)],
                        out.at[pl.ds(c * Npad + s * rpt, rpt)])

    return k


@functools.lru_cache(maxsize=None)
def _sc_gather(E, Npad, F, C, NB=2):
    Ew = E // _NW
    nk = Ew // C
    nrem = nk % NB
    nr = (nk - nrem) // NB

    @functools.partial(
        pl.kernel,
        mesh=_mesh(),
        out_type=(jax.ShapeDtypeStruct((E, F), _F32),
                  jax.ShapeDtypeStruct((E, F), _F32)),
        scratch_types=[
            pltpu.VMEM((nk, C), jnp.int32),
            pltpu.VMEM((nk, C), jnp.int32),
        ] + [pltpu.VMEM((C, F), _F32)] * (2 * NB)
          + [pltpu.SemaphoreType.DMA] * (4 * NB),
    )
    def k(ta, tb, row3, col3, gr, gc, idxr, idxc, *bufsems):
        bufr = bufsems[:NB]
        bufc = bufsems[NB:2 * NB]
        sems = bufsems[2 * NB:]
        grs, gcs, srs, scs = (sems[:NB], sems[NB:2 * NB],
                              sems[2 * NB:3 * NB], sems[3 * NB:])
        c = lax.axis_index("c")
        s = lax.axis_index("s")
        t = c * _NS + s
        pltpu.sync_copy(row3.at[t], idxr)
        pltpu.sync_copy(col3.at[t], idxc)
        for j in range(nrem):
            pltpu.sync_copy(ta.at[idxr.at[j]], bufr[0])
            pltpu.sync_copy(bufr[0], gr.at[pl.ds(t * Ew + j * C, C)])
            pltpu.sync_copy(tb.at[idxc.at[j]], bufc[0])
            pltpu.sync_copy(bufc[0], gc.at[pl.ds(t * Ew + j * C, C)])
        for b in range(NB):
            pltpu.async_copy(ta.at[idxr.at[nrem + b]], bufr[b], grs[b])
            pltpu.async_copy(tb.at[idxc.at[nrem + b]], bufc[b], gcs[b])

        def body(g, carry):
            for b in range(NB):
                kk = nrem + g * NB + b
                base = t * Ew + kk * C
                pltpu.make_async_copy(ta.at[idxr.at[kk]], bufr[b],
                                      grs[b]).wait()
                pltpu.make_async_copy(tb.at[idxc.at[kk]], bufc[b],
                                      gcs[b]).wait()
                pltpu.async_copy(bufr[b], gr.at[pl.ds(base, C)], srs[b])
                pltpu.async_copy(bufc[b], gc.at[pl.ds(base, C)], scs[b])
            for b in range(NB):
                kk = nrem + g * NB + b
                base = t * Ew + kk * C

                @pl.when(g + 1 < nr)
                def _():
                    pltpu.make_async_copy(bufr[b], gr.at[pl.ds(base, C)],
                                          srs[b]).wait()
                    pltpu.make_async_copy(bufc[b], gc.at[pl.ds(base, C)],
                                          scs[b]).wait()
                    pltpu.async_copy(ta.at[idxr.at[kk + NB]], bufr[b], grs[b])
                    pltpu.async_copy(tb.at[idxc.at[kk + NB]], bufc[b], gcs[b])
            return carry

        lax.fori_loop(0, nr, body, 0)
        for b in range(NB):
            kk = nk - NB + b
            base = t * Ew + kk * C
            pltpu.make_async_copy(bufr[b], gr.at[pl.ds(base, C)],
                                  srs[b]).wait()
            pltpu.make_async_copy(bufc[b], gc.at[pl.ds(base, C)],
                                  scs[b]).wait()

    return k


@functools.lru_cache(maxsize=None)
def _sc_scatter(E, Npad, F, C, NB=2):
    Ew = E // _NW
    nk = Ew // C
    nrem = nk % NB
    nr = (nk - nrem) // NB
    rpt = Npad // _NS

    @functools.partial(
        pl.kernel,
        mesh=_mesh(),
        out_type=jax.ShapeDtypeStruct((_NC * Npad, F), _F32),
        scratch_types=[
            pltpu.VMEM((nk, C), jnp.int32),
            pltpu.VMEM_SHARED((Npad, F), _F32),
        ] + [pltpu.VMEM((C, F), _F32)] * NB
          + [pltpu.SemaphoreType.DMA] * (2 * NB),
    )
    def k(eo, row3, zeros, out, idxv, acc, *bufsems):
        buf = bufsems[:NB]
        lds = bufsems[NB:2 * NB]
        scs = bufsems[2 * NB:]
        c = lax.axis_index("c")
        s = lax.axis_index("s")
        t = c * _NS + s
        pltpu.sync_copy(zeros, acc.at[pl.ds(s * rpt, rpt)])
        pltpu.sync_copy(row3.at[t], idxv)
        plsc.subcore_barrier()
        for j in range(nrem):
            pltpu.sync_copy(eo.at[pl.ds(t * Ew + j * C, C)], buf[0])
            pltpu.sync_copy(buf[0], acc.at[idxv.at[j]], add=True)
        for b in range(NB):
            pltpu.async_copy(eo.at[pl.ds(t * Ew + (nrem + b) * C, C)],
                             buf[b], lds[b])

        def body(g, carry):
            for b in range(NB):
                kk = nrem + g * NB + b
                base = t * Ew + kk * C
                pltpu.make_async_copy(eo.at[pl.ds(base, C)], buf[b],
                                      lds[b]).wait()
                pltpu.async_copy(buf[b], acc.at[idxv.at[kk]], scs[b],
                                 add=True)
            for b in range(NB):
                kk = nrem + g * NB + b

                @pl.when(g + 1 < nr)
                def _():
                    pltpu.make_async_copy(buf[b], acc.at[idxv.at[kk]],
                                          scs[b]).wait()
                    pltpu.async_copy(eo.at[pl.ds(t * Ew + (kk + NB) * C, C)],
                                     buf[b], lds[b])
            return carry

        lax.fori_loop(0, nr, body, 0)
        for b in range(NB):
            kk = nk - NB + b
            pltpu.make_async_copy(buf[b], acc.at[idxv.at[kk]], scs[b]).wait()
        plsc.subcore_barrier()
        pltpu.sync_copy(acc.at[pl.ds(s * rpt, rpt)],
                        out.at[pl.ds(c * Npad + s * rpt, rpt)])

    return k


@functools.lru_cache(maxsize=None)
def _tc_proj1(Npad, D, F, Bn):
    def body(x, wi, wj, ta, tb):
        xv = x[...]
        ta[...] = jnp.dot(xv, wi[...], preferred_element_type=_F32)
        tb[...] = jnp.dot(xv, wj[...], preferred_element_type=_F32)

    return pl.pallas_call(
        body,
        grid=(Npad // Bn,),
        in_specs=[
            pl.BlockSpec((Bn, D), lambda i: (i, 0)),
            pl.BlockSpec((D, F), lambda i: (0, 0)),
            pl.BlockSpec((D, F), lambda i: (0, 0)),
        ],
        out_specs=[pl.BlockSpec((Bn, F), lambda i: (i, 0))] * 2,
        out_shape=[jax.ShapeDtypeStruct((Npad, F), _F32)] * 2,
        compiler_params=pltpu.CompilerParams(
            dimension_semantics=("parallel",)),
    )


@functools.lru_cache(maxsize=None)
def _tc_node(Npad, F, CW, has_prev, Bn):
    nb = Npad // Bn

    def body(*refs):
        if has_prev:
            (p0, p1, c0, c1, xp, wi1, wi2, wj1, wj2, xo, ta, tb) = refs
        else:
            (p0, p1, c0, c1, wi1, wj1, xo, ta, tb) = refs
        cnt = c0[:, :1] + c1[:, :1]
        inv = 1.0 / jnp.maximum(cnt, 1.0)
        x = jnp.maximum((p0[...] + p1[...]) * inv, 0.0)
        xo[...] = x
        tav = jnp.dot(x, wi1[...], preferred_element_type=_F32)
        tbv = jnp.dot(x, wj1[...], preferred_element_type=_F32)
        if has_prev:
            xpv = xp[...]
            tav = tav + jnp.dot(xpv, wi2[...], preferred_element_type=_F32)
            tbv = tbv + jnp.dot(xpv, wj2[...], preferred_element_type=_F32)
        ta[...] = tav
        tb[...] = tbv

    bspec = pl.BlockSpec((Bn, F), lambda i: (i, 0))
    p0s = pl.BlockSpec((Bn, F), lambda i: (i, 0))
    p1s = pl.BlockSpec((Bn, F), lambda i: (nb + i, 0))
    c0s = pl.BlockSpec((Bn, CW), lambda i: (i, 0))
    c1s = pl.BlockSpec((Bn, CW), lambda i: (nb + i, 0))
    wspec = pl.BlockSpec((F, F), lambda i: (0, 0))
    in_specs = [p0s, p1s, c0s, c1s]
    if has_prev:
        in_specs += [bspec, wspec, wspec, wspec, wspec]
    else:
        in_specs += [wspec, wspec]
    return pl.pallas_call(
        body,
        grid=(nb,),
        in_specs=in_specs,
        out_specs=[bspec] * 3,
        out_shape=[jax.ShapeDtypeStruct((Npad, F), _F32)] * 3,
        compiler_params=pltpu.CompilerParams(
            dimension_semantics=("parallel",)),
    )


@functools.lru_cache(maxsize=None)
def _tc_node_final(Npad, F, CW, Bn):
    nb = Npad // Bn

    def body(p0, p1, c0, c1, xo):
        cnt = c0[:, :1] + c1[:, :1]
        inv = 1.0 / jnp.maximum(cnt, 1.0)
        xo[...] = jnp.maximum((p0[...] + p1[...]) * inv, 0.0)

    return pl.pallas_call(
        body,
        grid=(nb,),
        in_specs=[
            pl.BlockSpec((Bn, F), lambda i: (i, 0)),
            pl.BlockSpec((Bn, F), lambda i: (nb + i, 0)),
            pl.BlockSpec((Bn, CW), lambda i: (i, 0)),
            pl.BlockSpec((Bn, CW), lambda i: (nb + i, 0)),
        ],
        out_specs=pl.BlockSpec((Bn, F), lambda i: (i, 0)),
        out_shape=jax.ShapeDtypeStruct((Npad, F), _F32),
        compiler_params=pltpu.CompilerParams(
            dimension_semantics=("parallel",)),
    )


@functools.lru_cache(maxsize=None)
def _tc_edge(E, F, DE1, nparts, relu_e, final, Be):
    def body(*refs):
        i = 0
        gr, gc = refs[0], refs[1]
        i = 2
        e1 = refs[i]; i += 1
        e2 = refs[i] if nparts == 2 else None
        if nparts == 2:
            i += 1
        we1 = refs[i]; i += 1
        we2 = refs[i] if nparts == 2 else None
        if nparts == 2:
            i += 1
        ba, wb, bb = refs[i], refs[i + 1], refs[i + 2]
        i += 3
        if final:
            wl, bl = refs[i], refs[i + 1]
            i += 2
        eo_out = refs[i]; i += 1
        if final:
            e6_out, out_out = refs[i], refs[i + 1]

        a = gr[...] + gc[...] + ba[...]
        ev1 = e1[...]
        if relu_e:
            ev1 = jnp.maximum(ev1, 0.0)
        a = a + jnp.dot(ev1, we1[...], preferred_element_type=_F32)
        if nparts == 2:
            ev2 = jnp.maximum(e2[...], 0.0)
            a = a + jnp.dot(ev2, we2[...], preferred_element_type=_F32)
        h = jnp.maximum(a, 0.0)
        eo = jnp.dot(h, wb[...], preferred_element_type=_F32) + bb[...]
        eo_out[...] = eo
        if final:
            e6 = jnp.maximum(eo, 0.0)
            e6_out[...] = e6
            out_out[...] = jnp.dot(e6, wl[...],
                                   preferred_element_type=_F32) + bl[...]

    bspec = pl.BlockSpec((Be, F), lambda i: (i, 0))
    in_specs = [bspec, bspec, pl.BlockSpec((Be, DE1), lambda i: (i, 0))]
    if nparts == 2:
        in_specs.append(bspec)
    in_specs.append(pl.BlockSpec((DE1, F), lambda i: (0, 0)))
    if nparts == 2:
        in_specs.append(pl.BlockSpec((F, F), lambda i: (0, 0)))
    in_specs += [
        pl.BlockSpec((1, F), lambda i: (0, 0)),
        pl.BlockSpec((F, F), lambda i: (0, 0)),
        pl.BlockSpec((1, F), lambda i: (0, 0)),
    ]
    out_specs = [bspec]
    out_shape = [jax.ShapeDtypeStruct((E, F), _F32)]
    if final:
        in_specs += [
            pl.BlockSpec((F, 1), lambda i: (0, 0)),
            pl.BlockSpec((1, 1), lambda i: (0, 0)),
        ]
        out_specs += [bspec, pl.BlockSpec((Be, 1), lambda i: (i, 0))]
        out_shape += [jax.ShapeDtypeStruct((E, F), _F32),
                      jax.ShapeDtypeStruct((E, 1), _F32)]
    return pl.pallas_call(
        body,
        grid=(E // Be,),
        in_specs=in_specs,
        out_specs=out_specs,
        out_shape=out_shape,
        compiler_params=pltpu.CompilerParams(
            dimension_semantics=("parallel",)),
    )


def kernel(node_feat, edge_feat, edge_idx,
           W1a, b1a, W1b, b1b, W2a, b2a, W2b, b2b,
           W3a, b3a, W3b, b3b, W4a, b4a, W4b, b4b,
           W5a, b5a, W5b, b5b, W6a, b6a, W6b, b6b,
           Wl, bl):
    N, D = node_feat.shape
    E, DE = edge_feat.shape
    F = W1b.shape[0]
    Npad = -(-N // 1024) * 1024
    Bn = 1024
    Be = 2000
    Cg = 80
    NBg = 2
    Cs = 80
    NBs = 2
    CW = 128
    assert E % (_NW * Cg) == 0 and E % (_NW * Cs) == 0 and E % Be == 0
    assert Npad % (_NS * 8) == 0
    nkg = E // (_NW * Cg)
    nks = E // (_NW * Cs)

    row = edge_idx[0]
    col = edge_idx[1]
    row3g = row.reshape(_NW, nkg, Cg)
    col3g = col.reshape(_NW, nkg, Cg)
    row3s = row.reshape(_NW, nks, Cs)
    x0 = jnp.pad(node_feat, ((0, Npad - N), (0, 0)))
    rpt = Npad // _NS
    zeros_c = jnp.zeros((rpt, CW), _F32)
    zeros_f = jnp.zeros((rpt, F), _F32)
    ones_c = jnp.ones((Cs, CW), _F32)

    def gather(ta, tb, r3, c3):
        return _sc_gather(E, Npad, F, Cg, NBg)(ta, tb, r3, c3)

    def scatter(eo, r3, z):
        return _sc_scatter(E, Npad, F, Cs, NBs)(eo, r3, z)

    cnt = _sc_counts(E, Npad, Cs, CW)(row3s, ones_c, zeros_c)

    ta, tb = _tc_proj1(Npad, D, F, Bn)(x0, W1a[:D], W1a[D:2 * D])
    gr, gc = gather(ta, tb, row3g, col3g)
    eo1 = _tc_edge(E, F, DE, 1, False, False, Be)(
        gr, gc, edge_feat, W1a[2 * D:], b1a.reshape(1, F), W1b,
        b1b.reshape(1, F))[0]
    p = scatter(eo1, row3s, zeros_f)

    x1, ta, tb = _tc_node(Npad, F, CW, False, Bn)(
        p, p, cnt, cnt, W2a[:F], W2a[F:2 * F])
    gr, gc = gather(ta, tb, row3g, col3g)
    eo2 = _tc_edge(E, F, F, 1, True, False, Be)(
        gr, gc, eo1, W2a[2 * F:], b2a.reshape(1, F), W2b,
        b2b.reshape(1, F))[0]
    p = scatter(eo2, row3s, zeros_f)

    x_prev = x1
    eo_prev2, eo_prev = eo1, eo2
    e6 = outv = None
    for Wa, ba, Wb, bb in ((W3a, b3a, W3b, b3b), (W4a, b4a, W4b, b4b),
                           (W5a, b5a, W5b, b5b), (W6a, b6a, W6b, b6b)):
        final = Wa is W6a
        x_cur, ta, tb = _tc_node(Npad, F, CW, True, Bn)(
            p, p, cnt, cnt, x_prev,
            Wa[:F], Wa[F:2 * F], Wa[2 * F:3 * F], Wa[3 * F:4 * F])
        gr, gc = gather(ta, tb, row3g, col3g)
        edge_fn = _tc_edge(E, F, F, 2, True, final, Be)
        args = (gr, gc, eo_prev, eo_prev2, Wa[4 * F:5 * F], Wa[5 * F:6 * F],
                ba.reshape(1, F), Wb, bb.reshape(1, F))
        if final:
            eo, e6, outv = edge_fn(*args, Wl, bl.reshape(1, 1))
        else:
            eo = edge_fn(*args)[0]
        p = scatter(eo, row3s, zeros_f)
        x_prev, eo_prev2, eo_prev = x_cur, eo_prev, eo

    x6p = _tc_node_final(Npad, F, CW, Bn)(p, p, cnt, cnt)
    return (outv, x6p[:N], e6)


# speedup vs baseline: 3.8013x; 1.0036x over previous
import functools

import jax
import jax.numpy as jnp
from jax import lax
from jax.experimental import pallas as pl
from jax.experimental.pallas import tpu as pltpu
from jax.experimental.pallas import tpu_sc as plsc

_NC, _NS, _L = 2, 16, 16
_NW = _NC * _NS
_F32 = jnp.float32


def _mesh():
    return plsc.VectorSubcoreMesh(core_axis_name="c", subcore_axis_name="s")


@functools.lru_cache(maxsize=None)
def _sc_counts(E, Npad, C, W):
    Ew = E // _NW
    nk = Ew // C
    rpt = Npad // _NS

    @functools.partial(
        pl.kernel,
        mesh=_mesh(),
        out_type=jax.ShapeDtypeStruct((_NC * Npad, W), _F32),
        scratch_types=[
            pltpu.VMEM((nk, C), jnp.int32),
            pltpu.VMEM((C, W), _F32),
            pltpu.VMEM_SHARED((Npad, W), _F32),
        ],
    )
    def k(row3, ones, zeros, out, idxv, onesv, acc):
        c = lax.axis_index("c")
        s = lax.axis_index("s")
        t = c * _NS + s
        pltpu.sync_copy(zeros, acc.at[pl.ds(s * rpt, rpt)])
        pltpu.sync_copy(ones, onesv)
        pltpu.sync_copy(row3.at[t], idxv)
        plsc.subcore_barrier()

        def body(kk, carry):
            pltpu.sync_copy(onesv, acc.at[idxv.at[kk]], add=True)
            return carry

        lax.fori_loop(0, nk, body, 0)
        plsc.subcore_barrier()
        pltpu.sync_copy(acc.at[pl.ds(s * rpt, rpt)],
                        out.at[pl.ds(c * Npad + s * rpt, rpt)])

    return k


@functools.lru_cache(maxsize=None)
def _sc_gather(E, Npad, F, C, NB=2):
    Ew = E // _NW
    nk = Ew // C
    nrem = nk % NB
    nr = (nk - nrem) // NB

    @functools.partial(
        pl.kernel,
        mesh=_mesh(),
        out_type=(jax.ShapeDtypeStruct((E, F), _F32),
                  jax.ShapeDtypeStruct((E, F), _F32)),
        scratch_types=[
            pltpu.VMEM((nk, C), jnp.int32),
            pltpu.VMEM((nk, C), jnp.int32),
        ] + [pltpu.VMEM((C, F), _F32)] * (2 * NB)
          + [pltpu.SemaphoreType.DMA] * (4 * NB),
    )
    def k(ta, tb, row3, col3, gr, gc, idxr, idxc, *bufsems):
        bufr = bufsems[:NB]
        bufc = bufsems[NB:2 * NB]
        sems = bufsems[2 * NB:]
        grs, gcs, srs, scs = (sems[:NB], sems[NB:2 * NB],
                              sems[2 * NB:3 * NB], sems[3 * NB:])
        c = lax.axis_index("c")
        s = lax.axis_index("s")
        t = c * _NS + s
        pltpu.sync_copy(row3.at[t], idxr)
        pltpu.sync_copy(col3.at[t], idxc)
        for j in range(nrem):
            pltpu.sync_copy(ta.at[idxr.at[j]], bufr[0])
            pltpu.sync_copy(bufr[0], gr.at[pl.ds(t * Ew + j * C, C)])
            pltpu.sync_copy(tb.at[idxc.at[j]], bufc[0])
            pltpu.sync_copy(bufc[0], gc.at[pl.ds(t * Ew + j * C, C)])
        for b in range(NB):
            pltpu.async_copy(ta.at[idxr.at[nrem + b]], bufr[b], grs[b])
            pltpu.async_copy(tb.at[idxc.at[nrem + b]], bufc[b], gcs[b])

        def body(g, carry):
            for b in range(NB):
                kk = nrem + g * NB + b
                base = t * Ew + kk * C
                pltpu.make_async_copy(ta.at[idxr.at[kk]], bufr[b],
                                      grs[b]).wait()
                pltpu.make_async_copy(tb.at[idxc.at[kk]], bufc[b],
                                      gcs[b]).wait()
                pltpu.async_copy(bufr[b], gr.at[pl.ds(base, C)], srs[b])
                pltpu.async_copy(bufc[b], gc.at[pl.ds(base, C)], scs[b])
            for b in range(NB):
                kk = nrem + g * NB + b
                base = t * Ew + kk * C

                @pl.when(g + 1 < nr)
                def _():
                    pltpu.make_async_copy(bufr[b], gr.at[pl.ds(base, C)],
                                          srs[b]).wait()
                    pltpu.make_async_copy(bufc[b], gc.at[pl.ds(base, C)],
                                          scs[b]).wait()
                    pltpu.async_copy(ta.at[idxr.at[kk + NB]], bufr[b], grs[b])
                    pltpu.async_copy(tb.at[idxc.at[kk + NB]], bufc[b], gcs[b])
            return carry

        lax.fori_loop(0, nr, body, 0)
        for b in range(NB):
            kk = nk - NB + b
            base = t * Ew + kk * C
            pltpu.make_async_copy(bufr[b], gr.at[pl.ds(base, C)],
                                  srs[b]).wait()
            pltpu.make_async_copy(bufc[b], gc.at[pl.ds(base, C)],
                                  scs[b]).wait()

    return k


@functools.lru_cache(maxsize=None)
def _sc_scatter(E, Npad, F, C, NB=2):
    Ew = E // _NW
    nk = Ew // C
    nrem = nk % NB
    nr = (nk - nrem) // NB
    rpt = Npad // _NS

    @functools.partial(
        pl.kernel,
        mesh=_mesh(),
        out_type=jax.ShapeDtypeStruct((_NC * Npad, F), _F32),
        scratch_types=[
            pltpu.VMEM((nk, C), jnp.int32),
            pltpu.VMEM_SHARED((Npad, F), _F32),
        ] + [pltpu.VMEM((C, F), _F32)] * NB
          + [pltpu.SemaphoreType.DMA] * (2 * NB),
    )
    def k(eo, row3, zeros, out, idxv, acc, *bufsems):
        buf = bufsems[:NB]
        lds = bufsems[NB:2 * NB]
        scs = bufsems[2 * NB:]
        c = lax.axis_index("c")
        s = lax.axis_index("s")
        t = c * _NS + s
        pltpu.sync_copy(zeros, acc.at[pl.ds(s * rpt, rpt)])
        pltpu.sync_copy(row3.at[t], idxv)
        plsc.subcore_barrier()
        for j in range(nrem):
            pltpu.sync_copy(eo.at[pl.ds(t * Ew + j * C, C)], buf[0])
            pltpu.sync_copy(buf[0], acc.at[idxv.at[j]], add=True)
        for b in range(NB):
            pltpu.async_copy(eo.at[pl.ds(t * Ew + (nrem + b) * C, C)],
                             buf[b], lds[b])

        def body(g, carry):
            for b in range(NB):
                kk = nrem + g * NB + b
                base = t * Ew + kk * C
                pltpu.make_async_copy(eo.at[pl.ds(base, C)], buf[b],
                                      lds[b]).wait()
                pltpu.async_copy(buf[b], acc.at[idxv.at[kk]], scs[b],
                                 add=True)
            for b in range(NB):
                kk = nrem + g * NB + b

                @pl.when(g + 1 < nr)
                def _():
                    pltpu.make_async_copy(buf[b], acc.at[idxv.at[kk]],
                                          scs[b]).wait()
                    pltpu.async_copy(eo.at[pl.ds(t * Ew + (kk + NB) * C, C)],
                                     buf[b], lds[b])
            return carry

        lax.fori_loop(0, nr, body, 0)
        for b in range(NB):
            kk = nk - NB + b
            pltpu.make_async_copy(buf[b], acc.at[idxv.at[kk]], scs[b]).wait()
        plsc.subcore_barrier()
        pltpu.sync_copy(acc.at[pl.ds(s * rpt, rpt)],
                        out.at[pl.ds(c * Npad + s * rpt, rpt)])

    return k


@functools.lru_cache(maxsize=None)
def _tc_proj1(Npad, D, F, Bn):
    def body(x, wi, wj, ta, tb):
        xv = x[...]
        ta[...] = jnp.dot(xv, wi[...], preferred_element_type=_F32)
        tb[...] = jnp.dot(xv, wj[...], preferred_element_type=_F32)

    return pl.pallas_call(
        body,
        grid=(Npad // Bn,),
        in_specs=[
            pl.BlockSpec((Bn, D), lambda i: (i, 0)),
            pl.BlockSpec((D, F), lambda i: (0, 0)),
            pl.BlockSpec((D, F), lambda i: (0, 0)),
        ],
        out_specs=[pl.BlockSpec((Bn, F), lambda i: (i, 0))] * 2,
        out_shape=[jax.ShapeDtypeStruct((Npad, F), _F32)] * 2,
        compiler_params=pltpu.CompilerParams(
            dimension_semantics=("parallel",)),
    )


@functools.lru_cache(maxsize=None)
def _tc_node(Npad, F, CW, has_prev, Bn):
    nb = Npad // Bn

    def body(*refs):
        if has_prev:
            (p0, p1, p2, p3, c0, c1, xp, wi1, wi2, wj1, wj2, xo, ta, tb) = refs
        else:
            (p0, p1, p2, p3, c0, c1, wi1, wj1, xo, ta, tb) = refs
        cnt = c0[:, :1] + c1[:, :1]
        inv = 1.0 / jnp.maximum(cnt, 1.0)
        x = jnp.maximum((p0[...] + p1[...] + p2[...] + p3[...]) * inv, 0.0)
        xo[...] = x
        tav = jnp.dot(x, wi1[...], preferred_element_type=_F32)
        tbv = jnp.dot(x, wj1[...], preferred_element_type=_F32)
        if has_prev:
            xpv = xp[...]
            tav = tav + jnp.dot(xpv, wi2[...], preferred_element_type=_F32)
            tbv = tbv + jnp.dot(xpv, wj2[...], preferred_element_type=_F32)
        ta[...] = tav
        tb[...] = tbv

    bspec = pl.BlockSpec((Bn, F), lambda i: (i, 0))
    p0s = pl.BlockSpec((Bn, F), lambda i: (i, 0))
    p1s = pl.BlockSpec((Bn, F), lambda i: (nb + i, 0))
    c0s = pl.BlockSpec((Bn, CW), lambda i: (i, 0))
    c1s = pl.BlockSpec((Bn, CW), lambda i: (nb + i, 0))
    wspec = pl.BlockSpec((F, F), lambda i: (0, 0))
    in_specs = [p0s, p1s, p0s, p1s, c0s, c1s]
    if has_prev:
        in_specs += [bspec, wspec, wspec, wspec, wspec]
    else:
        in_specs += [wspec, wspec]
    return pl.pallas_call(
        body,
        grid=(nb,),
        in_specs=in_specs,
        out_specs=[bspec] * 3,
        out_shape=[jax.ShapeDtypeStruct((Npad, F), _F32)] * 3,
        compiler_params=pltpu.CompilerParams(
            dimension_semantics=("parallel",)),
    )


@functools.lru_cache(maxsize=None)
def _tc_node_final(Npad, F, CW, Bn):
    nb = Npad // Bn

    def body(p0, p1, p2, p3, c0, c1, xo):
        cnt = c0[:, :1] + c1[:, :1]
        inv = 1.0 / jnp.maximum(cnt, 1.0)
        xo[...] = jnp.maximum(
            (p0[...] + p1[...] + p2[...] + p3[...]) * inv, 0.0)

    p0s = pl.BlockSpec((Bn, F), lambda i: (i, 0))
    p1s = pl.BlockSpec((Bn, F), lambda i: (nb + i, 0))
    return pl.pallas_call(
        body,
        grid=(nb,),
        in_specs=[
            p0s, p1s, p0s, p1s,
            pl.BlockSpec((Bn, CW), lambda i: (i, 0)),
            pl.BlockSpec((Bn, CW), lambda i: (nb + i, 0)),
        ],
        out_specs=pl.BlockSpec((Bn, F), lambda i: (i, 0)),
        out_shape=jax.ShapeDtypeStruct((Npad, F), _F32),
        compiler_params=pltpu.CompilerParams(
            dimension_semantics=("parallel",)),
    )


@functools.lru_cache(maxsize=None)
def _tc_edge(E, F, DE1, nparts, relu_e, final, Be):
    def body(*refs):
        i = 0
        gr, gc = refs[0], refs[1]
        i = 2
        e1 = refs[i]; i += 1
        e2 = refs[i] if nparts == 2 else None
        if nparts == 2:
            i += 1
        we1 = refs[i]; i += 1
        we2 = refs[i] if nparts == 2 else None
        if nparts == 2:
            i += 1
        ba, wb, bb = refs[i], refs[i + 1], refs[i + 2]
        i += 3
        if final:
            wl, bl = refs[i], refs[i + 1]
            i += 2
        eo_out = refs[i]; i += 1
        if final:
            e6_out, out_out = refs[i], refs[i + 1]

        a = gr[...] + gc[...] + ba[...]
        ev1 = e1[...]
        if relu_e:
            ev1 = jnp.maximum(ev1, 0.0)
        a = a + jnp.dot(ev1, we1[...], preferred_element_type=_F32)
        if nparts == 2:
            ev2 = jnp.maximum(e2[...], 0.0)
            a = a + jnp.dot(ev2, we2[...], preferred_element_type=_F32)
        h = jnp.maximum(a, 0.0)
        eo = jnp.dot(h, wb[...], preferred_element_type=_F32) + bb[...]
        eo_out[...] = eo
        if final:
            e6 = jnp.maximum(eo, 0.0)
            e6_out[...] = e6
            out_out[...] = jnp.dot(e6, wl[...],
                                   preferred_element_type=_F32) + bl[...]

    bspec = pl.BlockSpec((Be, F), lambda i: (i, 0))
    in_specs = [bspec, bspec, pl.BlockSpec((Be, DE1), lambda i: (i, 0))]
    if nparts == 2:
        in_specs.append(bspec)
    in_specs.append(pl.BlockSpec((DE1, F), lambda i: (0, 0)))
    if nparts == 2:
        in_specs.append(pl.BlockSpec((F, F), lambda i: (0, 0)))
    in_specs += [
        pl.BlockSpec((1, F), lambda i: (0, 0)),
        pl.BlockSpec((F, F), lambda i: (0, 0)),
        pl.BlockSpec((1, F), lambda i: (0, 0)),
    ]
    out_specs = [bspec]
    out_shape = [jax.ShapeDtypeStruct((E, F), _F32)]
    if final:
        in_specs += [
            pl.BlockSpec((F, 1), lambda i: (0, 0)),
            pl.BlockSpec((1, 1), lambda i: (0, 0)),
        ]
        out_specs += [bspec, pl.BlockSpec((Be, 1), lambda i: (i, 0))]
        out_shape += [jax.ShapeDtypeStruct((E, F), _F32),
                      jax.ShapeDtypeStruct((E, 1), _F32)]
    return pl.pallas_call(
        body,
        grid=(E // Be,),
        in_specs=in_specs,
        out_specs=out_specs,
        out_shape=out_shape,
        compiler_params=pltpu.CompilerParams(
            dimension_semantics=("parallel",)),
    )


def kernel(node_feat, edge_feat, edge_idx,
           W1a, b1a, W1b, b1b, W2a, b2a, W2b, b2b,
           W3a, b3a, W3b, b3b, W4a, b4a, W4b, b4b,
           W5a, b5a, W5b, b5b, W6a, b6a, W6b, b6b,
           Wl, bl):
    N, D = node_feat.shape
    E, DE = edge_feat.shape
    F = W1b.shape[0]
    Npad = -(-N // 1024) * 1024
    Bn = 1024
    Be = 2000
    EH = E // 2
    EwH = EH // _NW
    Cg = 40
    Cs = 40
    NB = 2
    Cc = 80
    CW = 128
    assert EwH % Cg == 0 and EwH % Cs == 0 and EH % Be == 0
    assert E % (_NW * Cc) == 0 and Npad % (_NS * 8) == 0

    row = edge_idx[0]
    col = edge_idx[1]
    rowh = (row[:EH], row[EH:])
    colh = (col[:EH], col[EH:])
    r3g = [r.reshape(_NW, EwH // Cg, Cg) for r in rowh]
    c3g = [c.reshape(_NW, EwH // Cg, Cg) for c in colh]
    r3s = [r.reshape(_NW, EwH // Cs, Cs) for r in rowh]
    row3c = row.reshape(_NW, (E // _NW) // Cc, Cc)
    efh = (edge_feat[:EH], edge_feat[EH:])
    x0 = jnp.pad(node_feat, ((0, Npad - N), (0, 0)))
    rpt = Npad // _NS
    zeros_c = jnp.zeros((rpt, CW), _F32)
    zeros_f = jnp.zeros((rpt, F), _F32)
    ones_c = jnp.ones((Cc, CW), _F32)

    gat = _sc_gather(EH, Npad, F, Cg, NB)
    sca = _sc_scatter(EH, Npad, F, Cs, NB)

    cnt = _sc_counts(E, Npad, Cc, CW)(row3c, ones_c, zeros_c)

    ta, tb = _tc_proj1(Npad, D, F, Bn)(x0, W1a[:D], W1a[D:2 * D])
    edge1 = _tc_edge(EH, F, DE, 1, False, False, Be)
    g = [gat(ta, tb, r3g[h], c3g[h]) for h in (0, 1)]
    eo1 = [None, None]
    p = [None, None]
    for h in (0, 1):
        eo1[h] = edge1(g[h][0], g[h][1], efh[h], W1a[2 * D:],
                       b1a.reshape(1, F), W1b, b1b.reshape(1, F))[0]
        p[h] = sca(eo1[h], r3s[h], zeros_f)

    x1, ta, tb = _tc_node(Npad, F, CW, False, Bn)(
        p[0], p[0], p[1], p[1], cnt, cnt, W2a[:F], W2a[F:2 * F])
    edge2 = _tc_edge(EH, F, F, 1, True, False, Be)
    g = [gat(ta, tb, r3g[h], c3g[h]) for h in (0, 1)]
    eo2 = [None, None]
    for h in (0, 1):
        eo2[h] = edge2(g[h][0], g[h][1], eo1[h], W2a[2 * F:],
                       b2a.reshape(1, F), W2b, b2b.reshape(1, F))[0]
        p[h] = sca(eo2[h], r3s[h], zeros_f)

    x_prev = x1
    eo_prev2, eo_prev = eo1, eo2
    e6 = [None, None]
    outv = [None, None]
    for Wa, ba, Wb, bb in ((W3a, b3a, W3b, b3b), (W4a, b4a, W4b, b4b),
                           (W5a, b5a, W5b, b5b), (W6a, b6a, W6b, b6b)):
        final = Wa is W6a
        x_cur, ta, tb = _tc_node(Npad, F, CW, True, Bn)(
            p[0], p[0], p[1], p[1], cnt, cnt, x_prev,
            Wa[:F], Wa[F:2 * F], Wa[2 * F:3 * F], Wa[3 * F:4 * F])
        g = [gat(ta, tb, r3g[h], c3g[h]) for h in (0, 1)]
        edge_fn = _tc_edge(EH, F, F, 2, True, final, Be)
        eo = [None, None]
        for h in (0, 1):
            args = (g[h][0], g[h][1], eo_prev[h], eo_prev2[h],
                    Wa[4 * F:5 * F], Wa[5 * F:6 * F],
                    ba.reshape(1, F), Wb, bb.reshape(1, F))
            if final:
                eo[h], e6[h], outv[h] = edge_fn(*args, Wl, bl.reshape(1, 1))
            else:
                eo[h] = edge_fn(*args)[0]
            p[h] = sca(eo[h], r3s[h], zeros_f)
        x_prev, eo_prev2, eo_prev = x_cur, eo_prev, eo

    x6p = _tc_node_final(Npad, F, CW, Bn)(p[0], p[0], p[1], p[1], cnt, cnt)
    return (jnp.concatenate(outv), x6p[:N], jnp.concatenate(e6))


# speedup vs baseline: 3.9597x; 1.0417x over previous
import functools

import jax
import jax.numpy as jnp
from jax import lax
from jax.experimental import pallas as pl
from jax.experimental.pallas import tpu as pltpu
from jax.experimental.pallas import tpu_sc as plsc

_NC, _NS, _L = 2, 16, 16
_NW = _NC * _NS
_F32 = jnp.float32


def _mesh():
    return plsc.VectorSubcoreMesh(core_axis_name="c", subcore_axis_name="s")


@functools.lru_cache(maxsize=None)
def _sc_counts(E, Npad, C, W):
    Ew = E // _NW
    nk = Ew // C
    rpt = Npad // _NS

    @functools.partial(
        pl.kernel,
        mesh=_mesh(),
        out_type=jax.ShapeDtypeStruct((_NC * Npad, W), _F32),
        scratch_types=[
            pltpu.VMEM((nk, C), jnp.int32),
            pltpu.VMEM((C, W), _F32),
            pltpu.VMEM_SHARED((Npad, W), _F32),
        ],
    )
    def k(row3, ones, zeros, out, idxv, onesv, acc):
        c = lax.axis_index("c")
        s = lax.axis_index("s")
        t = c * _NS + s
        pltpu.sync_copy(zeros, acc.at[pl.ds(s * rpt, rpt)])
        pltpu.sync_copy(ones, onesv)
        pltpu.sync_copy(row3.at[t], idxv)
        plsc.subcore_barrier()

        def body(kk, carry):
            pltpu.sync_copy(onesv, acc.at[idxv.at[kk]], add=True)
            return carry

        lax.fori_loop(0, nk, body, 0)
        plsc.subcore_barrier()
        pltpu.sync_copy(acc.at[pl.ds(s * rpt, rpt)],
                        out.at[pl.ds(c * Npad + s * rpt, rpt)])

    return k


@functools.lru_cache(maxsize=None)
def _sc_gather(E, Npad, F, C, NB=2):
    Ew = E // _NW
    nk = Ew // C
    T = Ew - nk * C
    nrem = nk % NB
    nr = (nk - nrem) // NB

    @functools.partial(
        pl.kernel,
        mesh=_mesh(),
        out_type=(jax.ShapeDtypeStruct((E, F), _F32),
                  jax.ShapeDtypeStruct((E, F), _F32)),
        scratch_types=[
            pltpu.VMEM((nk, C), jnp.int32),
            pltpu.VMEM((nk, C), jnp.int32),
            pltpu.VMEM((1, max(T, 1)), jnp.int32),
            pltpu.VMEM((1, max(T, 1)), jnp.int32),
        ] + [pltpu.VMEM((C, F), _F32)] * (2 * NB)
          + [pltpu.SemaphoreType.DMA] * (4 * NB),
    )
    def k(ta, tb, row3, col3, rowt, colt, gr, gc, idxr, idxc, idxtr, idxtc,
          *bufsems):
        bufr = bufsems[:NB]
        bufc = bufsems[NB:2 * NB]
        sems = bufsems[2 * NB:]
        grs, gcs, srs, scs = (sems[:NB], sems[NB:2 * NB],
                              sems[2 * NB:3 * NB], sems[3 * NB:])
        c = lax.axis_index("c")
        s = lax.axis_index("s")
        t = c * _NS + s
        pltpu.sync_copy(row3.at[t], idxr)
        pltpu.sync_copy(col3.at[t], idxc)
        if T:
            tb_ = t * Ew + nk * C
            pltpu.sync_copy(rowt.at[pl.ds(t, 1)], idxtr)
            pltpu.sync_copy(colt.at[pl.ds(t, 1)], idxtc)
            pltpu.sync_copy(ta.at[idxtr.at[0]], bufr[0].at[pl.ds(0, T)])
            pltpu.sync_copy(bufr[0].at[pl.ds(0, T)], gr.at[pl.ds(tb_, T)])
            pltpu.sync_copy(tb.at[idxtc.at[0]], bufc[0].at[pl.ds(0, T)])
            pltpu.sync_copy(bufc[0].at[pl.ds(0, T)], gc.at[pl.ds(tb_, T)])
        for j in range(nrem):
            pltpu.sync_copy(ta.at[idxr.at[j]], bufr[0])
            pltpu.sync_copy(bufr[0], gr.at[pl.ds(t * Ew + j * C, C)])
            pltpu.sync_copy(tb.at[idxc.at[j]], bufc[0])
            pltpu.sync_copy(bufc[0], gc.at[pl.ds(t * Ew + j * C, C)])
        for b in range(NB):
            pltpu.async_copy(ta.at[idxr.at[nrem + b]], bufr[b], grs[b])
            pltpu.async_copy(tb.at[idxc.at[nrem + b]], bufc[b], gcs[b])

        def body(g, carry):
            for b in range(NB):
                kk = nrem + g * NB + b
                base = t * Ew + kk * C
                pltpu.make_async_copy(ta.at[idxr.at[kk]], bufr[b],
                                      grs[b]).wait()
                pltpu.make_async_copy(tb.at[idxc.at[kk]], bufc[b],
                                      gcs[b]).wait()
                pltpu.async_copy(bufr[b], gr.at[pl.ds(base, C)], srs[b])
                pltpu.async_copy(bufc[b], gc.at[pl.ds(base, C)], scs[b])
            for b in range(NB):
                kk = nrem + g * NB + b
                base = t * Ew + kk * C

                @pl.when(g + 1 < nr)
                def _():
                    pltpu.make_async_copy(bufr[b], gr.at[pl.ds(base, C)],
                                          srs[b]).wait()
                    pltpu.make_async_copy(bufc[b], gc.at[pl.ds(base, C)],
                                          scs[b]).wait()
                    pltpu.async_copy(ta.at[idxr.at[kk + NB]], bufr[b], grs[b])
                    pltpu.async_copy(tb.at[idxc.at[kk + NB]], bufc[b], gcs[b])
            return carry

        lax.fori_loop(0, nr, body, 0)
        for b in range(NB):
            kk = nk - NB + b
            base = t * Ew + kk * C
            pltpu.make_async_copy(bufr[b], gr.at[pl.ds(base, C)],
                                  srs[b]).wait()
            pltpu.make_async_copy(bufc[b], gc.at[pl.ds(base, C)],
                                  scs[b]).wait()

    return k


@functools.lru_cache(maxsize=None)
def _sc_scatter(E, Npad, F, C, NB=2):
    Ew = E // _NW
    nk = Ew // C
    T = Ew - nk * C
    nrem = nk % NB
    nr = (nk - nrem) // NB
    rpt = Npad // _NS

    @functools.partial(
        pl.kernel,
        mesh=_mesh(),
        out_type=jax.ShapeDtypeStruct((_NC * Npad, F), _F32),
        scratch_types=[
            pltpu.VMEM((nk, C), jnp.int32),
            pltpu.VMEM((1, max(T, 1)), jnp.int32),
            pltpu.VMEM_SHARED((Npad, F), _F32),
        ] + [pltpu.VMEM((C, F), _F32)] * NB
          + [pltpu.SemaphoreType.DMA] * (2 * NB),
    )
    def k(eo, row3, rowt, zeros, out, idxv, idxt, acc, *bufsems):
        buf = bufsems[:NB]
        lds = bufsems[NB:2 * NB]
        scs = bufsems[2 * NB:]
        c = lax.axis_index("c")
        s = lax.axis_index("s")
        t = c * _NS + s
        pltpu.sync_copy(zeros, acc.at[pl.ds(s * rpt, rpt)])
        pltpu.sync_copy(row3.at[t], idxv)
        plsc.subcore_barrier()
        if T:
            tb_ = t * Ew + nk * C
            pltpu.sync_copy(rowt.at[pl.ds(t, 1)], idxt)
            pltpu.sync_copy(eo.at[pl.ds(tb_, T)], buf[0].at[pl.ds(0, T)])
            pltpu.sync_copy(buf[0].at[pl.ds(0, T)], acc.at[idxt.at[0]],
                            add=True)
        for j in range(nrem):
            pltpu.sync_copy(eo.at[pl.ds(t * Ew + j * C, C)], buf[0])
            pltpu.sync_copy(buf[0], acc.at[idxv.at[j]], add=True)
        for b in range(NB):
            pltpu.async_copy(eo.at[pl.ds(t * Ew + (nrem + b) * C, C)],
                             buf[b], lds[b])

        def body(g, carry):
            for b in range(NB):
                kk = nrem + g * NB + b
                base = t * Ew + kk * C
                pltpu.make_async_copy(eo.at[pl.ds(base, C)], buf[b],
                                      lds[b]).wait()
                pltpu.async_copy(buf[b], acc.at[idxv.at[kk]], scs[b],
                                 add=True)
            for b in range(NB):
                kk = nrem + g * NB + b

                @pl.when(g + 1 < nr)
                def _():
                    pltpu.make_async_copy(buf[b], acc.at[idxv.at[kk]],
                                          scs[b]).wait()
                    pltpu.async_copy(eo.at[pl.ds(t * Ew + (kk + NB) * C, C)],
                                     buf[b], lds[b])
            return carry

        lax.fori_loop(0, nr, body, 0)
        for b in range(NB):
            kk = nk - NB + b
            pltpu.make_async_copy(buf[b], acc.at[idxv.at[kk]], scs[b]).wait()
        plsc.subcore_barrier()
        pltpu.sync_copy(acc.at[pl.ds(s * rpt, rpt)],
                        out.at[pl.ds(c * Npad + s * rpt, rpt)])

    return k


@functools.lru_cache(maxsize=None)
def _tc_proj1(Npad, D, F, Bn):
    def body(x, wi, wj, ta, tb):
        xv = x[...]
        ta[...] = jnp.dot(xv, wi[...], preferred_element_type=_F32)
        tb[...] = jnp.dot(xv, wj[...], preferred_element_type=_F32)

    return pl.pallas_call(
        body,
        grid=(Npad // Bn,),
        in_specs=[
            pl.BlockSpec((Bn, D), lambda i: (i, 0)),
            pl.BlockSpec((D, F), lambda i: (0, 0)),
            pl.BlockSpec((D, F), lambda i: (0, 0)),
        ],
        out_specs=[pl.BlockSpec((Bn, F), lambda i: (i, 0))] * 2,
        out_shape=[jax.ShapeDtypeStruct((Npad, F), _F32)] * 2,
        compiler_params=pltpu.CompilerParams(
            dimension_semantics=("parallel",)),
    )


@functools.lru_cache(maxsize=None)
def _tc_node(Npad, F, CW, has_prev, Bn):
    nb = Npad // Bn

    def body(*refs):
        if has_prev:
            (p0, p1, p2, p3, c0, c1, xp, wi1, wi2, wj1, wj2, xo, ta, tb) = refs
        else:
            (p0, p1, p2, p3, c0, c1, wi1, wj1, xo, ta, tb) = refs
        cnt = c0[:, :1] + c1[:, :1]
        inv = 1.0 / jnp.maximum(cnt, 1.0)
        x = jnp.maximum((p0[...] + p1[...] + p2[...] + p3[...]) * inv, 0.0)
        xo[...] = x
        tav = jnp.dot(x, wi1[...], preferred_element_type=_F32)
        tbv = jnp.dot(x, wj1[...], preferred_element_type=_F32)
        if has_prev:
            xpv = xp[...]
            tav = tav + jnp.dot(xpv, wi2[...], preferred_element_type=_F32)
            tbv = tbv + jnp.dot(xpv, wj2[...], preferred_element_type=_F32)
        ta[...] = tav
        tb[...] = tbv

    bspec = pl.BlockSpec((Bn, F), lambda i: (i, 0))
    p0s = pl.BlockSpec((Bn, F), lambda i: (i, 0))
    p1s = pl.BlockSpec((Bn, F), lambda i: (nb + i, 0))
    c0s = pl.BlockSpec((Bn, CW), lambda i: (i, 0))
    c1s = pl.BlockSpec((Bn, CW), lambda i: (nb + i, 0))
    wspec = pl.BlockSpec((F, F), lambda i: (0, 0))
    in_specs = [p0s, p1s, p0s, p1s, c0s, c1s]
    if has_prev:
        in_specs += [bspec, wspec, wspec, wspec, wspec]
    else:
        in_specs += [wspec, wspec]
    return pl.pallas_call(
        body,
        grid=(nb,),
        in_specs=in_specs,
        out_specs=[bspec] * 3,
        out_shape=[jax.ShapeDtypeStruct((Npad, F), _F32)] * 3,
        compiler_params=pltpu.CompilerParams(
            dimension_semantics=("parallel",)),
    )


@functools.lru_cache(maxsize=None)
def _tc_node_final(Npad, F, CW, Bn):
    nb = Npad // Bn

    def body(p0, p1, p2, p3, c0, c1, xo):
        cnt = c0[:, :1] + c1[:, :1]
        inv = 1.0 / jnp.maximum(cnt, 1.0)
        xo[...] = jnp.maximum(
            (p0[...] + p1[...] + p2[...] + p3[...]) * inv, 0.0)

    p0s = pl.BlockSpec((Bn, F), lambda i: (i, 0))
    p1s = pl.BlockSpec((Bn, F), lambda i: (nb + i, 0))
    return pl.pallas_call(
        body,
        grid=(nb,),
        in_specs=[
            p0s, p1s, p0s, p1s,
            pl.BlockSpec((Bn, CW), lambda i: (i, 0)),
            pl.BlockSpec((Bn, CW), lambda i: (nb + i, 0)),
        ],
        out_specs=pl.BlockSpec((Bn, F), lambda i: (i, 0)),
        out_shape=jax.ShapeDtypeStruct((Npad, F), _F32),
        compiler_params=pltpu.CompilerParams(
            dimension_semantics=("parallel",)),
    )


@functools.lru_cache(maxsize=None)
def _tc_edge(E, F, DE1, nparts, relu_e, final, Be):
    def body(*refs):
        i = 0
        gr, gc = refs[0], refs[1]
        i = 2
        e1 = refs[i]; i += 1
        e2 = refs[i] if nparts == 2 else None
        if nparts == 2:
            i += 1
        we1 = refs[i]; i += 1
        we2 = refs[i] if nparts == 2 else None
        if nparts == 2:
            i += 1
        ba, wb, bb = refs[i], refs[i + 1], refs[i + 2]
        i += 3
        if final:
            wl, bl = refs[i], refs[i + 1]
            i += 2
        eo_out = refs[i]; i += 1
        if final:
            e6_out, out_out = refs[i], refs[i + 1]

        a = gr[...] + gc[...] + ba[...]
        ev1 = e1[...]
        if relu_e:
            ev1 = jnp.maximum(ev1, 0.0)
        a = a + jnp.dot(ev1, we1[...], preferred_element_type=_F32)
        if nparts == 2:
            ev2 = jnp.maximum(e2[...], 0.0)
            a = a + jnp.dot(ev2, we2[...], preferred_element_type=_F32)
        h = jnp.maximum(a, 0.0)
        eo = jnp.dot(h, wb[...], preferred_element_type=_F32) + bb[...]
        eo_out[...] = eo
        if final:
            e6 = jnp.maximum(eo, 0.0)
            e6_out[...] = e6
            out_out[...] = jnp.dot(e6, wl[...],
                                   preferred_element_type=_F32) + bl[...]

    bspec = pl.BlockSpec((Be, F), lambda i: (i, 0))
    in_specs = [bspec, bspec, pl.BlockSpec((Be, DE1), lambda i: (i, 0))]
    if nparts == 2:
        in_specs.append(bspec)
    in_specs.append(pl.BlockSpec((DE1, F), lambda i: (0, 0)))
    if nparts == 2:
        in_specs.append(pl.BlockSpec((F, F), lambda i: (0, 0)))
    in_specs += [
        pl.BlockSpec((1, F), lambda i: (0, 0)),
        pl.BlockSpec((F, F), lambda i: (0, 0)),
        pl.BlockSpec((1, F), lambda i: (0, 0)),
    ]
    out_specs = [bspec]
    out_shape = [jax.ShapeDtypeStruct((E, F), _F32)]
    if final:
        in_specs += [
            pl.BlockSpec((F, 1), lambda i: (0, 0)),
            pl.BlockSpec((1, 1), lambda i: (0, 0)),
        ]
        out_specs += [bspec, pl.BlockSpec((Be, 1), lambda i: (i, 0))]
        out_shape += [jax.ShapeDtypeStruct((E, F), _F32),
                      jax.ShapeDtypeStruct((E, 1), _F32)]
    return pl.pallas_call(
        body,
        grid=(E // Be,),
        in_specs=in_specs,
        out_specs=out_specs,
        out_shape=out_shape,
        compiler_params=pltpu.CompilerParams(
            dimension_semantics=("parallel",)),
    )


def kernel(node_feat, edge_feat, edge_idx,
           W1a, b1a, W1b, b1b, W2a, b2a, W2b, b2b,
           W3a, b3a, W3b, b3b, W4a, b4a, W4b, b4b,
           W5a, b5a, W5b, b5b, W6a, b6a, W6b, b6b,
           Wl, bl):
    N, D = node_feat.shape
    E, DE = edge_feat.shape
    F = W1b.shape[0]
    Npad = -(-N // 1024) * 1024
    Bn = 1024
    Be = 2000
    EH = E // 2
    EwH = EH // _NW
    Cg = 128
    Cs = 128
    NB = 2
    Cc = 80
    CW = 128
    nkg = EwH // Cg
    Tg = EwH - nkg * Cg
    assert EH % Be == 0 and Cg % 8 == 0 and Tg % 8 == 0 and Cg <= 128
    assert E % (_NW * Cc) == 0 and Npad % (_NS * 8) == 0

    row = edge_idx[0]
    col = edge_idx[1]
    rowh = [r.reshape(_NW, EwH) for r in (row[:EH], row[EH:])]
    colh = [c.reshape(_NW, EwH) for c in (col[:EH], col[EH:])]
    r3g = [r[:, :nkg * Cg].reshape(_NW, nkg, Cg) for r in rowh]
    c3g = [c[:, :nkg * Cg].reshape(_NW, nkg, Cg) for c in colh]
    rtg = [r[:, nkg * Cg:] for r in rowh]
    ctg = [c[:, nkg * Cg:] for c in colh]
    row3c = row.reshape(_NW, (E // _NW) // Cc, Cc)
    efh = (edge_feat[:EH], edge_feat[EH:])
    x0 = jnp.pad(node_feat, ((0, Npad - N), (0, 0)))
    rpt = Npad // _NS
    zeros_c = jnp.zeros((rpt, CW), _F32)
    zeros_f = jnp.zeros((rpt, F), _F32)
    ones_c = jnp.ones((Cc, CW), _F32)

    gat = _sc_gather(EH, Npad, F, Cg, NB)
    sca = _sc_scatter(EH, Npad, F, Cs, NB)

    cnt = _sc_counts(E, Npad, Cc, CW)(row3c, ones_c, zeros_c)

    ta, tb = _tc_proj1(Npad, D, F, Bn)(x0, W1a[:D], W1a[D:2 * D])
    edge1 = _tc_edge(EH, F, DE, 1, False, False, Be)
    g = [gat(ta, tb, r3g[h], c3g[h], rtg[h], ctg[h]) for h in (0, 1)]
    eo1 = [None, None]
    p = [None, None]
    for h in (0, 1):
        eo1[h] = edge1(g[h][0], g[h][1], efh[h], W1a[2 * D:],
                       b1a.reshape(1, F), W1b, b1b.reshape(1, F))[0]
        p[h] = sca(eo1[h], r3g[h], rtg[h], zeros_f)

    x1, ta, tb = _tc_node(Npad, F, CW, False, Bn)(
        p[0], p[0], p[1], p[1], cnt, cnt, W2a[:F], W2a[F:2 * F])
    edge2 = _tc_edge(EH, F, F, 1, True, False, Be)
    g = [gat(ta, tb, r3g[h], c3g[h], rtg[h], ctg[h]) for h in (0, 1)]
    eo2 = [None, None]
    for h in (0, 1):
        eo2[h] = edge2(g[h][0], g[h][1], eo1[h], W2a[2 * F:],
                       b2a.reshape(1, F), W2b, b2b.reshape(1, F))[0]
        p[h] = sca(eo2[h], r3g[h], rtg[h], zeros_f)

    x_prev = x1
    eo_prev2, eo_prev = eo1, eo2
    e6 = [None, None]
    outv = [None, None]
    for Wa, ba, Wb, bb in ((W3a, b3a, W3b, b3b), (W4a, b4a, W4b, b4b),
                           (W5a, b5a, W5b, b5b), (W6a, b6a, W6b, b6b)):
        final = Wa is W6a
        x_cur, ta, tb = _tc_node(Npad, F, CW, True, Bn)(
            p[0], p[0], p[1], p[1], cnt, cnt, x_prev,
            Wa[:F], Wa[F:2 * F], Wa[2 * F:3 * F], Wa[3 * F:4 * F])
        g = [gat(ta, tb, r3g[h], c3g[h], rtg[h], ctg[h]) for h in (0, 1)]
        edge_fn = _tc_edge(EH, F, F, 2, True, final, Be)
        eo = [None, None]
        for h in (0, 1):
            args = (g[h][0], g[h][1], eo_prev[h], eo_prev2[h],
                    Wa[4 * F:5 * F], Wa[5 * F:6 * F],
                    ba.reshape(1, F), Wb, bb.reshape(1, F))
            if final:
                eo[h], e6[h], outv[h] = edge_fn(*args, Wl, bl.reshape(1, 1))
            else:
                eo[h] = edge_fn(*args)[0]
            p[h] = sca(eo[h], r3g[h], rtg[h], zeros_f)
        x_prev, eo_prev2, eo_prev = x_cur, eo_prev, eo

    x6p = _tc_node_final(Npad, F, CW, Bn)(p[0], p[0], p[1], p[1], cnt, cnt)
    return (jnp.concatenate(outv), x6p[:N], jnp.concatenate(e6))


# speedup vs baseline: 3.9618x; 1.0005x over previous
import functools

import jax
import jax.numpy as jnp
from jax import lax
from jax.experimental import pallas as pl
from jax.experimental.pallas import tpu as pltpu
from jax.experimental.pallas import tpu_sc as plsc

_NC, _NS, _L = 2, 16, 16
_NW = _NC * _NS
_F32 = jnp.float32


def _mesh():
    return plsc.VectorSubcoreMesh(core_axis_name="c", subcore_axis_name="s")


@functools.lru_cache(maxsize=None)
def _sc_counts(E, Npad, C, W):
    Ew = E // _NW
    nk = Ew // C
    rpt = Npad // _NS

    @functools.partial(
        pl.kernel,
        mesh=_mesh(),
        out_type=jax.ShapeDtypeStruct((_NC * Npad, W), _F32),
        scratch_types=[
            pltpu.VMEM((nk, C), jnp.int32),
            pltpu.VMEM((C, W), _F32),
            pltpu.VMEM_SHARED((Npad, W), _F32),
        ],
    )
    def k(row3, ones, zeros, out, idxv, onesv, acc):
        c = lax.axis_index("c")
        s = lax.axis_index("s")
        t = c * _NS + s
        pltpu.sync_copy(zeros, acc.at[pl.ds(s * rpt, rpt)])
        pltpu.sync_copy(ones, onesv)
        pltpu.sync_copy(row3.at[t], idxv)
        plsc.subcore_barrier()

        def body(kk, carry):
            pltpu.sync_copy(onesv, acc.at[idxv.at[kk]], add=True)
            return carry

        lax.fori_loop(0, nk, body, 0)
        plsc.subcore_barrier()
        pltpu.sync_copy(acc.at[pl.ds(s * rpt, rpt)],
                        out.at[pl.ds(c * Npad + s * rpt, rpt)])

    return k


@functools.lru_cache(maxsize=None)
def _sc_gather(E, Npad, F, C, NB=2):
    Ew = E // _NW
    nk = Ew // C
    T = Ew - nk * C
    nrem = nk % NB
    nr = (nk - nrem) // NB

    @functools.partial(
        pl.kernel,
        mesh=_mesh(),
        out_type=(jax.ShapeDtypeStruct((E, F), _F32),
                  jax.ShapeDtypeStruct((E, F), _F32)),
        scratch_types=[
            pltpu.VMEM((nk, C), jnp.int32),
            pltpu.VMEM((nk, C), jnp.int32),
            pltpu.VMEM((1, max(T, 1)), jnp.int32),
            pltpu.VMEM((1, max(T, 1)), jnp.int32),
        ] + [pltpu.VMEM((C, F), _F32)] * (2 * NB)
          + [pltpu.SemaphoreType.DMA] * (4 * NB),
    )
    def k(ta, tb, row3, col3, rowt, colt, gr, gc, idxr, idxc, idxtr, idxtc,
          *bufsems):
        bufr = bufsems[:NB]
        bufc = bufsems[NB:2 * NB]
        sems = bufsems[2 * NB:]
        grs, gcs, srs, scs = (sems[:NB], sems[NB:2 * NB],
                              sems[2 * NB:3 * NB], sems[3 * NB:])
        c = lax.axis_index("c")
        s = lax.axis_index("s")
        t = c * _NS + s
        pltpu.sync_copy(row3.at[t], idxr)
        pltpu.sync_copy(col3.at[t], idxc)
        if T:
            tb_ = t * Ew + nk * C
            pltpu.sync_copy(rowt.at[pl.ds(t, 1)], idxtr)
            pltpu.sync_copy(colt.at[pl.ds(t, 1)], idxtc)
            pltpu.sync_copy(ta.at[idxtr.at[0]], bufr[0].at[pl.ds(0, T)])
            pltpu.sync_copy(bufr[0].at[pl.ds(0, T)], gr.at[pl.ds(tb_, T)])
            pltpu.sync_copy(tb.at[idxtc.at[0]], bufc[0].at[pl.ds(0, T)])
            pltpu.sync_copy(bufc[0].at[pl.ds(0, T)], gc.at[pl.ds(tb_, T)])
        for j in range(nrem):
            pltpu.sync_copy(ta.at[idxr.at[j]], bufr[0])
            pltpu.sync_copy(bufr[0], gr.at[pl.ds(t * Ew + j * C, C)])
            pltpu.sync_copy(tb.at[idxc.at[j]], bufc[0])
            pltpu.sync_copy(bufc[0], gc.at[pl.ds(t * Ew + j * C, C)])
        for b in range(NB):
            pltpu.async_copy(ta.at[idxr.at[nrem + b]], bufr[b], grs[b])
            pltpu.async_copy(tb.at[idxc.at[nrem + b]], bufc[b], gcs[b])

        def body(g, carry):
            for b in range(NB):
                kk = nrem + g * NB + b
                base = t * Ew + kk * C
                pltpu.make_async_copy(ta.at[idxr.at[kk]], bufr[b],
                                      grs[b]).wait()
                pltpu.async_copy(bufr[b], gr.at[pl.ds(base, C)], srs[b])
                pltpu.make_async_copy(tb.at[idxc.at[kk]], bufc[b],
                                      gcs[b]).wait()
                pltpu.async_copy(bufc[b], gc.at[pl.ds(base, C)], scs[b])
            for b in range(NB):
                kk = nrem + g * NB + b
                base = t * Ew + kk * C

                @pl.when(g + 1 < nr)
                def _():
                    pltpu.make_async_copy(bufr[b], gr.at[pl.ds(base, C)],
                                          srs[b]).wait()
                    pltpu.make_async_copy(bufc[b], gc.at[pl.ds(base, C)],
                                          scs[b]).wait()
                    pltpu.async_copy(ta.at[idxr.at[kk + NB]], bufr[b], grs[b])
                    pltpu.async_copy(tb.at[idxc.at[kk + NB]], bufc[b], gcs[b])
            return carry

        lax.fori_loop(0, nr, body, 0)
        for b in range(NB):
            kk = nk - NB + b
            base = t * Ew + kk * C
            pltpu.make_async_copy(bufr[b], gr.at[pl.ds(base, C)],
                                  srs[b]).wait()
            pltpu.make_async_copy(bufc[b], gc.at[pl.ds(base, C)],
                                  scs[b]).wait()

    return k


@functools.lru_cache(maxsize=None)
def _sc_scatter(E, Npad, F, C, NB=2):
    Ew = E // _NW
    nk = Ew // C
    T = Ew - nk * C
    nrem = nk % NB
    nr = (nk - nrem) // NB
    rpt = Npad // _NS

    @functools.partial(
        pl.kernel,
        mesh=_mesh(),
        out_type=jax.ShapeDtypeStruct((_NC * Npad, F), _F32),
        scratch_types=[
            pltpu.VMEM((nk, C), jnp.int32),
            pltpu.VMEM((1, max(T, 1)), jnp.int32),
            pltpu.VMEM_SHARED((Npad, F), _F32),
        ] + [pltpu.VMEM((C, F), _F32)] * NB
          + [pltpu.SemaphoreType.DMA] * (2 * NB),
    )
    def k(eo, row3, rowt, zeros, out, idxv, idxt, acc, *bufsems):
        buf = bufsems[:NB]
        lds = bufsems[NB:2 * NB]
        scs = bufsems[2 * NB:]
        c = lax.axis_index("c")
        s = lax.axis_index("s")
        t = c * _NS + s
        pltpu.sync_copy(zeros, acc.at[pl.ds(s * rpt, rpt)])
        pltpu.sync_copy(row3.at[t], idxv)
        plsc.subcore_barrier()
        if T:
            tb_ = t * Ew + nk * C
            pltpu.sync_copy(rowt.at[pl.ds(t, 1)], idxt)
            pltpu.sync_copy(eo.at[pl.ds(tb_, T)], buf[0].at[pl.ds(0, T)])
            pltpu.sync_copy(buf[0].at[pl.ds(0, T)], acc.at[idxt.at[0]],
                            add=True)
        for j in range(nrem):
            pltpu.sync_copy(eo.at[pl.ds(t * Ew + j * C, C)], buf[0])
            pltpu.sync_copy(buf[0], acc.at[idxv.at[j]], add=True)
        for b in range(NB):
            pltpu.async_copy(eo.at[pl.ds(t * Ew + (nrem + b) * C, C)],
                             buf[b], lds[b])

        def body(g, carry):
            for b in range(NB):
                kk = nrem + g * NB + b
                base = t * Ew + kk * C
                pltpu.make_async_copy(eo.at[pl.ds(base, C)], buf[b],
                                      lds[b]).wait()
                pltpu.async_copy(buf[b], acc.at[idxv.at[kk]], scs[b],
                                 add=True)
            for b in range(NB):
                kk = nrem + g * NB + b

                @pl.when(g + 1 < nr)
                def _():
                    pltpu.make_async_copy(buf[b], acc.at[idxv.at[kk]],
                                          scs[b]).wait()
                    pltpu.async_copy(eo.at[pl.ds(t * Ew + (kk + NB) * C, C)],
                                     buf[b], lds[b])
            return carry

        lax.fori_loop(0, nr, body, 0)
        for b in range(NB):
            kk = nk - NB + b
            pltpu.make_async_copy(buf[b], acc.at[idxv.at[kk]], scs[b]).wait()
        plsc.subcore_barrier()
        pltpu.sync_copy(acc.at[pl.ds(s * rpt, rpt)],
                        out.at[pl.ds(c * Npad + s * rpt, rpt)])

    return k


@functools.lru_cache(maxsize=None)
def _tc_proj1(Npad, D, F, Bn):
    def body(x, wi, wj, ta, tb):
        xv = x[...]
        ta[...] = jnp.dot(xv, wi[...], preferred_element_type=_F32)
        tb[...] = jnp.dot(xv, wj[...], preferred_element_type=_F32)

    return pl.pallas_call(
        body,
        grid=(Npad // Bn,),
        in_specs=[
            pl.BlockSpec((Bn, D), lambda i: (i, 0)),
            pl.BlockSpec((D, F), lambda i: (0, 0)),
            pl.BlockSpec((D, F), lambda i: (0, 0)),
        ],
        out_specs=[pl.BlockSpec((Bn, F), lambda i: (i, 0))] * 2,
        out_shape=[jax.ShapeDtypeStruct((Npad, F), _F32)] * 2,
        compiler_params=pltpu.CompilerParams(
            dimension_semantics=("parallel",)),
    )


@functools.lru_cache(maxsize=None)
def _tc_node(Npad, F, CW, has_prev, Bn):
    nb = Npad // Bn

    def body(*refs):
        if has_prev:
            (p0, p1, p2, p3, c0, c1, xp, wi1, wi2, wj1, wj2, xo, ta, tb) = refs
        else:
            (p0, p1, p2, p3, c0, c1, wi1, wj1, xo, ta, tb) = refs
        cnt = c0[:, :1] + c1[:, :1]
        inv = 1.0 / jnp.maximum(cnt, 1.0)
        x = jnp.maximum((p0[...] + p1[...] + p2[...] + p3[...]) * inv, 0.0)
        xo[...] = x
        tav = jnp.dot(x, wi1[...], preferred_element_type=_F32)
        tbv = jnp.dot(x, wj1[...], preferred_element_type=_F32)
        if has_prev:
            xpv = xp[...]
            tav = tav + jnp.dot(xpv, wi2[...], preferred_element_type=_F32)
            tbv = tbv + jnp.dot(xpv, wj2[...], preferred_element_type=_F32)
        ta[...] = tav
        tb[...] = tbv

    bspec = pl.BlockSpec((Bn, F), lambda i: (i, 0))
    p0s = pl.BlockSpec((Bn, F), lambda i: (i, 0))
    p1s = pl.BlockSpec((Bn, F), lambda i: (nb + i, 0))
    c0s = pl.BlockSpec((Bn, CW), lambda i: (i, 0))
    c1s = pl.BlockSpec((Bn, CW), lambda i: (nb + i, 0))
    wspec = pl.BlockSpec((F, F), lambda i: (0, 0))
    in_specs = [p0s, p1s, p0s, p1s, c0s, c1s]
    if has_prev:
        in_specs += [bspec, wspec, wspec, wspec, wspec]
    else:
        in_specs += [wspec, wspec]
    return pl.pallas_call(
        body,
        grid=(nb,),
        in_specs=in_specs,
        out_specs=[bspec] * 3,
        out_shape=[jax.ShapeDtypeStruct((Npad, F), _F32)] * 3,
        compiler_params=pltpu.CompilerParams(
            dimension_semantics=("parallel",)),
    )


@functools.lru_cache(maxsize=None)
def _tc_node_final(Npad, F, CW, Bn):
    nb = Npad // Bn

    def body(p0, p1, p2, p3, c0, c1, xo):
        cnt = c0[:, :1] + c1[:, :1]
        inv = 1.0 / jnp.maximum(cnt, 1.0)
        xo[...] = jnp.maximum(
            (p0[...] + p1[...] + p2[...] + p3[...]) * inv, 0.0)

    p0s = pl.BlockSpec((Bn, F), lambda i: (i, 0))
    p1s = pl.BlockSpec((Bn, F), lambda i: (nb + i, 0))
    return pl.pallas_call(
        body,
        grid=(nb,),
        in_specs=[
            p0s, p1s, p0s, p1s,
            pl.BlockSpec((Bn, CW), lambda i: (i, 0)),
            pl.BlockSpec((Bn, CW), lambda i: (nb + i, 0)),
        ],
        out_specs=pl.BlockSpec((Bn, F), lambda i: (i, 0)),
        out_shape=jax.ShapeDtypeStruct((Npad, F), _F32),
        compiler_params=pltpu.CompilerParams(
            dimension_semantics=("parallel",)),
    )


@functools.lru_cache(maxsize=None)
def _tc_edge(E, F, DE1, nparts, relu_e, final, Be):
    def body(*refs):
        i = 0
        gr, gc = refs[0], refs[1]
        i = 2
        e1 = refs[i]; i += 1
        e2 = refs[i] if nparts == 2 else None
        if nparts == 2:
            i += 1
        we1 = refs[i]; i += 1
        we2 = refs[i] if nparts == 2 else None
        if nparts == 2:
            i += 1
        ba, wb, bb = refs[i], refs[i + 1], refs[i + 2]
        i += 3
        if final:
            wl, bl = refs[i], refs[i + 1]
            i += 2
        eo_out = refs[i]; i += 1
        if final:
            e6_out, out_out = refs[i], refs[i + 1]

        a = gr[...] + gc[...] + ba[...]
        ev1 = e1[...]
        if relu_e:
            ev1 = jnp.maximum(ev1, 0.0)
        a = a + jnp.dot(ev1, we1[...], preferred_element_type=_F32)
        if nparts == 2:
            ev2 = jnp.maximum(e2[...], 0.0)
            a = a + jnp.dot(ev2, we2[...], preferred_element_type=_F32)
        h = jnp.maximum(a, 0.0)
        eo = jnp.dot(h, wb[...], preferred_element_type=_F32) + bb[...]
        eo_out[...] = eo
        if final:
            e6 = jnp.maximum(eo, 0.0)
            e6_out[...] = e6
            out_out[...] = jnp.dot(e6, wl[...],
                                   preferred_element_type=_F32) + bl[...]

    bspec = pl.BlockSpec((Be, F), lambda i: (i, 0))
    in_specs = [bspec, bspec, pl.BlockSpec((Be, DE1), lambda i: (i, 0))]
    if nparts == 2:
        in_specs.append(bspec)
    in_specs.append(pl.BlockSpec((DE1, F), lambda i: (0, 0)))
    if nparts == 2:
        in_specs.append(pl.BlockSpec((F, F), lambda i: (0, 0)))
    in_specs += [
        pl.BlockSpec((1, F), lambda i: (0, 0)),
        pl.BlockSpec((F, F), lambda i: (0, 0)),
        pl.BlockSpec((1, F), lambda i: (0, 0)),
    ]
    out_specs = [bspec]
    out_shape = [jax.ShapeDtypeStruct((E, F), _F32)]
    if final:
        in_specs += [
            pl.BlockSpec((F, 1), lambda i: (0, 0)),
            pl.BlockSpec((1, 1), lambda i: (0, 0)),
        ]
        out_specs += [bspec, pl.BlockSpec((Be, 1), lambda i: (i, 0))]
        out_shape += [jax.ShapeDtypeStruct((E, F), _F32),
                      jax.ShapeDtypeStruct((E, 1), _F32)]
    return pl.pallas_call(
        body,
        grid=(E // Be,),
        in_specs=in_specs,
        out_specs=out_specs,
        out_shape=out_shape,
        compiler_params=pltpu.CompilerParams(
            dimension_semantics=("parallel",)),
    )


def kernel(node_feat, edge_feat, edge_idx,
           W1a, b1a, W1b, b1b, W2a, b2a, W2b, b2b,
           W3a, b3a, W3b, b3b, W4a, b4a, W4b, b4b,
           W5a, b5a, W5b, b5b, W6a, b6a, W6b, b6b,
           Wl, bl):
    N, D = node_feat.shape
    E, DE = edge_feat.shape
    F = W1b.shape[0]
    Npad = -(-N // 1024) * 1024
    Bn = 1024
    Be = 2000
    EH = E // 2
    EwH = EH // _NW
    Cg = 128
    Cs = 128
    NB = 2
    Cc = 80
    CW = 128
    nkg = EwH // Cg
    Tg = EwH - nkg * Cg
    assert EH % Be == 0 and Cg % 8 == 0 and Tg % 8 == 0 and Cg <= 128
    assert E % (_NW * Cc) == 0 and Npad % (_NS * 8) == 0

    row = edge_idx[0]
    col = edge_idx[1]
    rowh = [r.reshape(_NW, EwH) for r in (row[:EH], row[EH:])]
    colh = [c.reshape(_NW, EwH) for c in (col[:EH], col[EH:])]
    r3g = [r[:, :nkg * Cg].reshape(_NW, nkg, Cg) for r in rowh]
    c3g = [c[:, :nkg * Cg].reshape(_NW, nkg, Cg) for c in colh]
    rtg = [r[:, nkg * Cg:] for r in rowh]
    ctg = [c[:, nkg * Cg:] for c in colh]
    row3c = row.reshape(_NW, (E // _NW) // Cc, Cc)
    efh = (edge_feat[:EH], edge_feat[EH:])
    x0 = jnp.pad(node_feat, ((0, Npad - N), (0, 0)))
    rpt = Npad // _NS
    zeros_c = jnp.zeros((rpt, CW), _F32)
    zeros_f = jnp.zeros((rpt, F), _F32)
    ones_c = jnp.ones((Cc, CW), _F32)

    gat = _sc_gather(EH, Npad, F, Cg, NB)
    sca = _sc_scatter(EH, Npad, F, Cs, NB)

    cnt = _sc_counts(E, Npad, Cc, CW)(row3c, ones_c, zeros_c)

    ta, tb = _tc_proj1(Npad, D, F, Bn)(x0, W1a[:D], W1a[D:2 * D])
    edge1 = _tc_edge(EH, F, DE, 1, False, False, Be)
    g = [gat(ta, tb, r3g[h], c3g[h], rtg[h], ctg[h]) for h in (0, 1)]
    eo1 = [None, None]
    p = [None, None]
    for h in (0, 1):
        eo1[h] = edge1(g[h][0], g[h][1], efh[h], W1a[2 * D:],
                       b1a.reshape(1, F), W1b, b1b.reshape(1, F))[0]
        p[h] = sca(eo1[h], r3g[h], rtg[h], zeros_f)

    x1, ta, tb = _tc_node(Npad, F, CW, False, Bn)(
        p[0], p[0], p[1], p[1], cnt, cnt, W2a[:F], W2a[F:2 * F])
    edge2 = _tc_edge(EH, F, F, 1, True, False, Be)
    g = [gat(ta, tb, r3g[h], c3g[h], rtg[h], ctg[h]) for h in (0, 1)]
    eo2 = [None, None]
    for h in (0, 1):
        eo2[h] = edge2(g[h][0], g[h][1], eo1[h], W2a[2 * F:],
                       b2a.reshape(1, F), W2b, b2b.reshape(1, F))[0]
        p[h] = sca(eo2[h], r3g[h], rtg[h], zeros_f)

    x_prev = x1
    eo_prev2, eo_prev = eo1, eo2
    e6 = [None, None]
    outv = [None, None]
    for Wa, ba, Wb, bb in ((W3a, b3a, W3b, b3b), (W4a, b4a, W4b, b4b),
                           (W5a, b5a, W5b, b5b), (W6a, b6a, W6b, b6b)):
        final = Wa is W6a
        x_cur, ta, tb = _tc_node(Npad, F, CW, True, Bn)(
            p[0], p[0], p[1], p[1], cnt, cnt, x_prev,
            Wa[:F], Wa[F:2 * F], Wa[2 * F:3 * F], Wa[3 * F:4 * F])
        g = [gat(ta, tb, r3g[h], c3g[h], rtg[h], ctg[h]) for h in (0, 1)]
        edge_fn = _tc_edge(EH, F, F, 2, True, final, Be)
        eo = [None, None]
        for h in (0, 1):
            args = (g[h][0], g[h][1], eo_prev[h], eo_prev2[h],
                    Wa[4 * F:5 * F], Wa[5 * F:6 * F],
                    ba.reshape(1, F), Wb, bb.reshape(1, F))
            if final:
                eo[h], e6[h], outv[h] = edge_fn(*args, Wl, bl.reshape(1, 1))
            else:
                eo[h] = edge_fn(*args)[0]
            p[h] = sca(eo[h], r3g[h], rtg[h], zeros_f)
        x_prev, eo_prev2, eo_prev = x_cur, eo_prev, eo

    x6p = _tc_node_final(Npad, F, CW, Bn)(p[0], p[0], p[1], p[1], cnt, cnt)
    return (jnp.concatenate(outv), x6p[:N], jnp.concatenate(e6))


# speedup vs baseline: 4.0496x; 1.0221x over previous
import functools

import jax
import jax.numpy as jnp
from jax import lax
from jax.experimental import pallas as pl
from jax.experimental.pallas import tpu as pltpu
from jax.experimental.pallas import tpu_sc as plsc

_NC, _NS, _L = 2, 16, 16
_NW = _NC * _NS
_F32 = jnp.float32


def _mesh():
    return plsc.VectorSubcoreMesh(core_axis_name="c", subcore_axis_name="s")


@functools.lru_cache(maxsize=None)
def _sc_counts(E, Npad, C, W):
    Ew = E // _NW
    nk = Ew // C
    rpt = Npad // _NS

    @functools.partial(
        pl.kernel,
        mesh=_mesh(),
        out_type=jax.ShapeDtypeStruct((_NC * Npad, W), _F32),
        scratch_types=[
            pltpu.VMEM((nk, C), jnp.int32),
            pltpu.VMEM((C, W), _F32),
            pltpu.VMEM_SHARED((Npad, W), _F32),
        ],
    )
    def k(row3, ones, zeros, out, idxv, onesv, acc):
        c = lax.axis_index("c")
        s = lax.axis_index("s")
        t = c * _NS + s
        pltpu.sync_copy(zeros, acc.at[pl.ds(s * rpt, rpt)])
        pltpu.sync_copy(ones, onesv)
        pltpu.sync_copy(row3.at[t], idxv)
        plsc.subcore_barrier()

        def body(kk, carry):
            pltpu.sync_copy(onesv, acc.at[idxv.at[kk]], add=True)
            return carry

        lax.fori_loop(0, nk, body, 0)
        plsc.subcore_barrier()
        pltpu.sync_copy(acc.at[pl.ds(s * rpt, rpt)],
                        out.at[pl.ds(c * Npad + s * rpt, rpt)])

    return k


@functools.lru_cache(maxsize=None)
def _sc_gather(E, Npad, F, C, NB=2):
    Ew = E // _NW
    nk = Ew // C
    T = Ew - nk * C
    nrem = nk % NB
    nr = (nk - nrem) // NB

    @functools.partial(
        pl.kernel,
        mesh=_mesh(),
        out_type=(jax.ShapeDtypeStruct((E, F), _F32),
                  jax.ShapeDtypeStruct((E, F), _F32)),
        scratch_types=[
            pltpu.VMEM((nk, C), jnp.int32),
            pltpu.VMEM((nk, C), jnp.int32),
            pltpu.VMEM((1, max(T, 1)), jnp.int32),
            pltpu.VMEM((1, max(T, 1)), jnp.int32),
        ] + [pltpu.VMEM((C, F), _F32)] * (2 * NB)
          + [pltpu.SemaphoreType.DMA] * (4 * NB),
    )
    def k(ta, tb, row3, col3, rowt, colt, gr, gc, idxr, idxc, idxtr, idxtc,
          *bufsems):
        bufr = bufsems[:NB]
        bufc = bufsems[NB:2 * NB]
        sems = bufsems[2 * NB:]
        grs, gcs, srs, scs = (sems[:NB], sems[NB:2 * NB],
                              sems[2 * NB:3 * NB], sems[3 * NB:])
        c = lax.axis_index("c")
        s = lax.axis_index("s")
        t = c * _NS + s
        pltpu.sync_copy(row3.at[t], idxr)
        pltpu.sync_copy(col3.at[t], idxc)
        if T:
            tb_ = t * Ew + nk * C
            pltpu.sync_copy(rowt.at[pl.ds(t, 1)], idxtr)
            pltpu.sync_copy(colt.at[pl.ds(t, 1)], idxtc)
            pltpu.sync_copy(ta.at[idxtr.at[0]], bufr[0].at[pl.ds(0, T)])
            pltpu.sync_copy(bufr[0].at[pl.ds(0, T)], gr.at[pl.ds(tb_, T)])
            pltpu.sync_copy(tb.at[idxtc.at[0]], bufc[0].at[pl.ds(0, T)])
            pltpu.sync_copy(bufc[0].at[pl.ds(0, T)], gc.at[pl.ds(tb_, T)])
        for j in range(nrem):
            pltpu.sync_copy(ta.at[idxr.at[j]], bufr[0])
            pltpu.sync_copy(bufr[0], gr.at[pl.ds(t * Ew + j * C, C)])
            pltpu.sync_copy(tb.at[idxc.at[j]], bufc[0])
            pltpu.sync_copy(bufc[0], gc.at[pl.ds(t * Ew + j * C, C)])
        for b in range(NB):
            pltpu.async_copy(ta.at[idxr.at[nrem + b]], bufr[b], grs[b])
            pltpu.async_copy(tb.at[idxc.at[nrem + b]], bufc[b], gcs[b])

        def body(g, carry):
            for b in range(NB):
                kk = nrem + g * NB + b
                base = t * Ew + kk * C
                pltpu.make_async_copy(ta.at[idxr.at[kk]], bufr[b],
                                      grs[b]).wait()
                pltpu.async_copy(bufr[b], gr.at[pl.ds(base, C)], srs[b])
                pltpu.make_async_copy(tb.at[idxc.at[kk]], bufc[b],
                                      gcs[b]).wait()
                pltpu.async_copy(bufc[b], gc.at[pl.ds(base, C)], scs[b])
            for b in range(NB):
                kk = nrem + g * NB + b
                base = t * Ew + kk * C

                @pl.when(g + 1 < nr)
                def _():
                    pltpu.make_async_copy(bufr[b], gr.at[pl.ds(base, C)],
                                          srs[b]).wait()
                    pltpu.make_async_copy(bufc[b], gc.at[pl.ds(base, C)],
                                          scs[b]).wait()
                    pltpu.async_copy(ta.at[idxr.at[kk + NB]], bufr[b], grs[b])
                    pltpu.async_copy(tb.at[idxc.at[kk + NB]], bufc[b], gcs[b])
            return carry

        lax.fori_loop(0, nr, body, 0)
        for b in range(NB):
            kk = nk - NB + b
            base = t * Ew + kk * C
            pltpu.make_async_copy(bufr[b], gr.at[pl.ds(base, C)],
                                  srs[b]).wait()
            pltpu.make_async_copy(bufc[b], gc.at[pl.ds(base, C)],
                                  scs[b]).wait()

    return k


@functools.lru_cache(maxsize=None)
def _sc_scatter(E, Npad, F, C, NB=2):
    Ew = E // _NW
    nk = Ew // C
    T = Ew - nk * C
    nrem = nk % NB
    nr = (nk - nrem) // NB
    rpt = Npad // _NS

    @functools.partial(
        pl.kernel,
        mesh=_mesh(),
        out_type=jax.ShapeDtypeStruct((_NC * Npad, F), _F32),
        scratch_types=[
            pltpu.VMEM((nk, C), jnp.int32),
            pltpu.VMEM((1, max(T, 1)), jnp.int32),
            pltpu.VMEM_SHARED((Npad, F), _F32),
        ] + [pltpu.VMEM((C, F), _F32)] * NB
          + [pltpu.SemaphoreType.DMA] * (2 * NB),
    )
    def k(eo, row3, rowt, zeros, out, idxv, idxt, acc, *bufsems):
        buf = bufsems[:NB]
        lds = bufsems[NB:2 * NB]
        scs = bufsems[2 * NB:]
        c = lax.axis_index("c")
        s = lax.axis_index("s")
        t = c * _NS + s
        pltpu.sync_copy(zeros, acc.at[pl.ds(s * rpt, rpt)])
        pltpu.sync_copy(row3.at[t], idxv)
        plsc.subcore_barrier()
        if T:
            tb_ = t * Ew + nk * C
            pltpu.sync_copy(rowt.at[pl.ds(t, 1)], idxt)
            pltpu.sync_copy(eo.at[pl.ds(tb_, T)], buf[0].at[pl.ds(0, T)])
            pltpu.sync_copy(buf[0].at[pl.ds(0, T)], acc.at[idxt.at[0]],
                            add=True)
        for j in range(nrem):
            pltpu.sync_copy(eo.at[pl.ds(t * Ew + j * C, C)], buf[0])
            pltpu.sync_copy(buf[0], acc.at[idxv.at[j]], add=True)
        for b in range(NB):
            pltpu.async_copy(eo.at[pl.ds(t * Ew + (nrem + b) * C, C)],
                             buf[b], lds[b])

        def body(g, carry):
            for b in range(NB):
                kk = nrem + g * NB + b
                base = t * Ew + kk * C
                pltpu.make_async_copy(eo.at[pl.ds(base, C)], buf[b],
                                      lds[b]).wait()
                pltpu.async_copy(buf[b], acc.at[idxv.at[kk]], scs[b],
                                 add=True)
            for b in range(NB):
                kk = nrem + g * NB + b

                @pl.when(g + 1 < nr)
                def _():
                    pltpu.make_async_copy(buf[b], acc.at[idxv.at[kk]],
                                          scs[b]).wait()
                    pltpu.async_copy(eo.at[pl.ds(t * Ew + (kk + NB) * C, C)],
                                     buf[b], lds[b])
            return carry

        lax.fori_loop(0, nr, body, 0)
        for b in range(NB):
            kk = nk - NB + b
            pltpu.make_async_copy(buf[b], acc.at[idxv.at[kk]], scs[b]).wait()
        plsc.subcore_barrier()
        pltpu.sync_copy(acc.at[pl.ds(s * rpt, rpt)],
                        out.at[pl.ds(c * Npad + s * rpt, rpt)])

    return k


@functools.lru_cache(maxsize=None)
def _tc_proj1(Npad, D, F, Bn):
    def body(x, wi, wj, ta, tb):
        xv = x[...]
        ta[...] = jnp.dot(xv, wi[...], preferred_element_type=_F32)
        tb[...] = jnp.dot(xv, wj[...], preferred_element_type=_F32)

    return pl.pallas_call(
        body,
        grid=(Npad // Bn,),
        in_specs=[
            pl.BlockSpec((Bn, D), lambda i: (i, 0)),
            pl.BlockSpec((D, F), lambda i: (0, 0)),
            pl.BlockSpec((D, F), lambda i: (0, 0)),
        ],
        out_specs=[pl.BlockSpec((Bn, F), lambda i: (i, 0))] * 2,
        out_shape=[jax.ShapeDtypeStruct((Npad, F), _F32)] * 2,
        compiler_params=pltpu.CompilerParams(
            dimension_semantics=("parallel",)),
    )


@functools.lru_cache(maxsize=None)
def _tc_node(Npad, F, CW, has_prev, Bn):
    nb = Npad // Bn

    def body(*refs):
        if has_prev:
            (p0, p1, p2, p3, c0, c1, xp, wi1, wi2, wj1, wj2, xo, ta, tb) = refs
        else:
            (p0, p1, p2, p3, c0, c1, wi1, wj1, xo, ta, tb) = refs
        cnt = c0[:, :1] + c1[:, :1]
        inv = 1.0 / jnp.maximum(cnt, 1.0)
        x = jnp.maximum((p0[...] + p1[...] + p2[...] + p3[...]) * inv, 0.0)
        xo[...] = x
        tav = jnp.dot(x, wi1[...], preferred_element_type=_F32)
        tbv = jnp.dot(x, wj1[...], preferred_element_type=_F32)
        if has_prev:
            xpv = xp[...]
            tav = tav + jnp.dot(xpv, wi2[...], preferred_element_type=_F32)
            tbv = tbv + jnp.dot(xpv, wj2[...], preferred_element_type=_F32)
        ta[...] = tav
        tb[...] = tbv

    bspec = pl.BlockSpec((Bn, F), lambda i: (i, 0))
    p0s = pl.BlockSpec((Bn, F), lambda i: (i, 0))
    p1s = pl.BlockSpec((Bn, F), lambda i: (nb + i, 0))
    c0s = pl.BlockSpec((Bn, CW), lambda i: (i, 0))
    c1s = pl.BlockSpec((Bn, CW), lambda i: (nb + i, 0))
    wspec = pl.BlockSpec((F, F), lambda i: (0, 0))
    in_specs = [p0s, p1s, p0s, p1s, c0s, c1s]
    if has_prev:
        in_specs += [bspec, wspec, wspec, wspec, wspec]
    else:
        in_specs += [wspec, wspec]
    return pl.pallas_call(
        body,
        grid=(nb,),
        in_specs=in_specs,
        out_specs=[bspec] * 3,
        out_shape=[jax.ShapeDtypeStruct((Npad, F), _F32)] * 3,
        compiler_params=pltpu.CompilerParams(
            dimension_semantics=("parallel",)),
    )


@functools.lru_cache(maxsize=None)
def _tc_node_final(Npad, F, CW, Bn):
    nb = Npad // Bn

    def body(p0, p1, p2, p3, c0, c1, xo):
        cnt = c0[:, :1] + c1[:, :1]
        inv = 1.0 / jnp.maximum(cnt, 1.0)
        xo[...] = jnp.maximum(
            (p0[...] + p1[...] + p2[...] + p3[...]) * inv, 0.0)

    p0s = pl.BlockSpec((Bn, F), lambda i: (i, 0))
    p1s = pl.BlockSpec((Bn, F), lambda i: (nb + i, 0))
    return pl.pallas_call(
        body,
        grid=(nb,),
        in_specs=[
            p0s, p1s, p0s, p1s,
            pl.BlockSpec((Bn, CW), lambda i: (i, 0)),
            pl.BlockSpec((Bn, CW), lambda i: (nb + i, 0)),
        ],
        out_specs=pl.BlockSpec((Bn, F), lambda i: (i, 0)),
        out_shape=jax.ShapeDtypeStruct((Npad, F), _F32),
        compiler_params=pltpu.CompilerParams(
            dimension_semantics=("parallel",)),
    )


@functools.lru_cache(maxsize=None)
def _tc_edge(E, F, DE1, nparts, relu_e, final, Be):
    def body(*refs):
        i = 0
        gr, gc = refs[0], refs[1]
        i = 2
        e1 = refs[i]; i += 1
        e2 = refs[i] if nparts == 2 else None
        if nparts == 2:
            i += 1
        we1 = refs[i]; i += 1
        we2 = refs[i] if nparts == 2 else None
        if nparts == 2:
            i += 1
        ba, wb, bb = refs[i], refs[i + 1], refs[i + 2]
        i += 3
        if final:
            wl, bl = refs[i], refs[i + 1]
            i += 2
        eo_out = refs[i]; i += 1
        if final:
            e6_out, out_out = refs[i], refs[i + 1]

        a = gr[...] + gc[...] + ba[...]
        ev1 = e1[...]
        if relu_e:
            ev1 = jnp.maximum(ev1, 0.0)
        a = a + jnp.dot(ev1, we1[...], preferred_element_type=_F32)
        if nparts == 2:
            ev2 = jnp.maximum(e2[...], 0.0)
            a = a + jnp.dot(ev2, we2[...], preferred_element_type=_F32)
        h = jnp.maximum(a, 0.0)
        eo = jnp.dot(h, wb[...], preferred_element_type=_F32) + bb[...]
        eo_out[...] = eo
        if final:
            e6 = jnp.maximum(eo, 0.0)
            e6_out[...] = e6
            out_out[...] = jnp.dot(e6, wl[...],
                                   preferred_element_type=_F32) + bl[...]

    bspec = pl.BlockSpec((Be, F), lambda i: (i, 0))
    in_specs = [bspec, bspec, pl.BlockSpec((Be, DE1), lambda i: (i, 0))]
    if nparts == 2:
        in_specs.append(bspec)
    in_specs.append(pl.BlockSpec((DE1, F), lambda i: (0, 0)))
    if nparts == 2:
        in_specs.append(pl.BlockSpec((F, F), lambda i: (0, 0)))
    in_specs += [
        pl.BlockSpec((1, F), lambda i: (0, 0)),
        pl.BlockSpec((F, F), lambda i: (0, 0)),
        pl.BlockSpec((1, F), lambda i: (0, 0)),
    ]
    out_specs = [bspec]
    out_shape = [jax.ShapeDtypeStruct((E, F), _F32)]
    if final:
        in_specs += [
            pl.BlockSpec((F, 1), lambda i: (0, 0)),
            pl.BlockSpec((1, 1), lambda i: (0, 0)),
        ]
        out_specs += [bspec, pl.BlockSpec((Be, 1), lambda i: (i, 0))]
        out_shape += [jax.ShapeDtypeStruct((E, F), _F32),
                      jax.ShapeDtypeStruct((E, 1), _F32)]
    return pl.pallas_call(
        body,
        grid=(E // Be,),
        in_specs=in_specs,
        out_specs=out_specs,
        out_shape=out_shape,
        compiler_params=pltpu.CompilerParams(
            dimension_semantics=("parallel",)),
    )


def kernel(node_feat, edge_feat, edge_idx,
           W1a, b1a, W1b, b1b, W2a, b2a, W2b, b2b,
           W3a, b3a, W3b, b3b, W4a, b4a, W4b, b4b,
           W5a, b5a, W5b, b5b, W6a, b6a, W6b, b6b,
           Wl, bl):
    N, D = node_feat.shape
    E, DE = edge_feat.shape
    F = W1b.shape[0]
    Npad = -(-N // 1024) * 1024
    Bn = 1024
    Be = 4000
    EH = E // 2
    EwH = EH // _NW
    Cg = 128
    Cs = 128
    NB = 2
    Cc = 80
    CW = 128
    nkg = EwH // Cg
    Tg = EwH - nkg * Cg
    assert EH % Be == 0 and Cg % 8 == 0 and Tg % 8 == 0 and Cg <= 128
    assert E % (_NW * Cc) == 0 and Npad % (_NS * 8) == 0

    row = edge_idx[0]
    col = edge_idx[1]
    rowh = [r.reshape(_NW, EwH) for r in (row[:EH], row[EH:])]
    colh = [c.reshape(_NW, EwH) for c in (col[:EH], col[EH:])]
    r3g = [r[:, :nkg * Cg].reshape(_NW, nkg, Cg) for r in rowh]
    c3g = [c[:, :nkg * Cg].reshape(_NW, nkg, Cg) for c in colh]
    rtg = [r[:, nkg * Cg:] for r in rowh]
    ctg = [c[:, nkg * Cg:] for c in colh]
    row3c = row.reshape(_NW, (E // _NW) // Cc, Cc)
    efh = (edge_feat[:EH], edge_feat[EH:])
    x0 = jnp.pad(node_feat, ((0, Npad - N), (0, 0)))
    rpt = Npad // _NS
    zeros_c = jnp.zeros((rpt, CW), _F32)
    zeros_f = jnp.zeros((rpt, F), _F32)
    ones_c = jnp.ones((Cc, CW), _F32)

    gat = _sc_gather(EH, Npad, F, Cg, NB)
    sca = _sc_scatter(EH, Npad, F, Cs, NB)

    cnt = _sc_counts(E, Npad, Cc, CW)(row3c, ones_c, zeros_c)

    ta, tb = _tc_proj1(Npad, D, F, Bn)(x0, W1a[:D], W1a[D:2 * D])
    edge1 = _tc_edge(EH, F, DE, 1, False, False, Be)
    g = [gat(ta, tb, r3g[h], c3g[h], rtg[h], ctg[h]) for h in (0, 1)]
    eo1 = [None, None]
    p = [None, None]
    for h in (0, 1):
        eo1[h] = edge1(g[h][0], g[h][1], efh[h], W1a[2 * D:],
                       b1a.reshape(1, F), W1b, b1b.reshape(1, F))[0]
        p[h] = sca(eo1[h], r3g[h], rtg[h], zeros_f)

    x1, ta, tb = _tc_node(Npad, F, CW, False, Bn)(
        p[0], p[0], p[1], p[1], cnt, cnt, W2a[:F], W2a[F:2 * F])
    edge2 = _tc_edge(EH, F, F, 1, True, False, Be)
    g = [gat(ta, tb, r3g[h], c3g[h], rtg[h], ctg[h]) for h in (0, 1)]
    eo2 = [None, None]
    for h in (0, 1):
        eo2[h] = edge2(g[h][0], g[h][1], eo1[h], W2a[2 * F:],
                       b2a.reshape(1, F), W2b, b2b.reshape(1, F))[0]
        p[h] = sca(eo2[h], r3g[h], rtg[h], zeros_f)

    x_prev = x1
    eo_prev2, eo_prev = eo1, eo2
    e6 = [None, None]
    outv = [None, None]
    for Wa, ba, Wb, bb in ((W3a, b3a, W3b, b3b), (W4a, b4a, W4b, b4b),
                           (W5a, b5a, W5b, b5b), (W6a, b6a, W6b, b6b)):
        final = Wa is W6a
        x_cur, ta, tb = _tc_node(Npad, F, CW, True, Bn)(
            p[0], p[0], p[1], p[1], cnt, cnt, x_prev,
            Wa[:F], Wa[F:2 * F], Wa[2 * F:3 * F], Wa[3 * F:4 * F])
        g = [gat(ta, tb, r3g[h], c3g[h], rtg[h], ctg[h]) for h in (0, 1)]
        edge_fn = _tc_edge(EH, F, F, 2, True, final, Be)
        eo = [None, None]
        for h in (0, 1):
            args = (g[h][0], g[h][1], eo_prev[h], eo_prev2[h],
                    Wa[4 * F:5 * F], Wa[5 * F:6 * F],
                    ba.reshape(1, F), Wb, bb.reshape(1, F))
            if final:
                eo[h], e6[h], outv[h] = edge_fn(*args, Wl, bl.reshape(1, 1))
            else:
                eo[h] = edge_fn(*args)[0]
            p[h] = sca(eo[h], r3g[h], rtg[h], zeros_f)
        x_prev, eo_prev2, eo_prev = x_cur, eo_prev, eo

    x6p = _tc_node_final(Npad, F, CW, Bn)(p[0], p[0], p[1], p[1], cnt, cnt)
    return (jnp.concatenate(outv), x6p[:N], jnp.concatenate(e6))
